# baseline (speedup 1.0000x reference)
import jax
import jax.numpy as jnp
from jax import lax
from jax.experimental import pallas as pl
from jax.experimental.pallas import tpu as pltpu
from jax.experimental.pallas import tpu_sc as plsc

F32 = jnp.float32

_N = 10000
_E = 320000
_H = 128
_L = 4

_EBLK = 8000
_NBLK = 2000
_NW = 32
_EW = _E // _NW
_CH = 80
_NCH = _EW // _CH
_ES = _E // 16
_NP = 10240
_NRS = _NP // 16
_ZR = 128


def _mm_bias(x, w, b, blk):
    m = x.shape[0]

    def body(x_ref, w_ref, b_ref, o_ref):
        o_ref[...] = (
            jnp.dot(x_ref[...], w_ref[...], preferred_element_type=F32)
            + b_ref[...]
        )

    return pl.pallas_call(
        body,
        grid=(m // blk,),
        in_specs=[
            pl.BlockSpec((blk, _H), lambda i: (i, 0)),
            pl.BlockSpec((_H, _H), lambda i: (0, 0)),
            pl.BlockSpec((1, _H), lambda i: (0, 0)),
        ],
        out_specs=pl.BlockSpec((blk, _H), lambda i: (i, 0)),
        out_shape=jax.ShapeDtypeStruct((m, _H), F32),
    )(x, w, b.reshape(1, _H))


def _node_mm(hh, w4):
    def body(h_ref, w_ref, o_ref):
        for k in range(4):
            o_ref[k] = jnp.dot(h_ref[...], w_ref[k], preferred_element_type=F32)

    return pl.pallas_call(
        body,
        grid=(_N // _NBLK,),
        in_specs=[
            pl.BlockSpec((_NBLK, _H), lambda i: (i, 0)),
            pl.BlockSpec((4, _H, _H), lambda i: (0, 0, 0)),
        ],
        out_specs=pl.BlockSpec((4, _NBLK, _H), lambda i: (0, i, 0)),
        out_shape=jax.ShapeDtypeStruct((4, _N, _H), F32),
    )(hh, w4)


def _edge_a(ee, g, c):
    def body(ee_ref, g_ref, c_ref, en_ref, s1_ref, s2_ref):
        en = (
            jnp.dot(ee_ref[...], c_ref[...], preferred_element_type=F32)
            + g_ref[...]
        )
        en_ref[...] = en

        @pl.when(pl.program_id(0) == 0)
        def _():
            s1_ref[...] = jnp.zeros_like(s1_ref)
            s2_ref[...] = jnp.zeros_like(s2_ref)

        s1_ref[...] += jnp.broadcast_to(
            jnp.sum(en, axis=0, keepdims=True), (8, _H))
        s2_ref[...] += jnp.broadcast_to(
            jnp.sum(en * en, axis=0, keepdims=True), (8, _H))

    return pl.pallas_call(
        body,
        grid=(_E // _EBLK,),
        in_specs=[
            pl.BlockSpec((_EBLK, _H), lambda i: (i, 0)),
            pl.BlockSpec((_EBLK, _H), lambda i: (i, 0)),
            pl.BlockSpec((_H, _H), lambda i: (0, 0)),
        ],
        out_specs=[
            pl.BlockSpec((_EBLK, _H), lambda i: (i, 0)),
            pl.BlockSpec((8, _H), lambda i: (0, 0)),
            pl.BlockSpec((8, _H), lambda i: (0, 0)),
        ],
        out_shape=[
            jax.ShapeDtypeStruct((_E, _H), F32),
            jax.ShapeDtypeStruct((8, _H), F32),
            jax.ShapeDtypeStruct((8, _H), F32),
        ],
    )(ee, g, c)


def _edge_b(en, ee, s1, s2, gam, bet):
    def body(en_ref, ee_ref, s1_ref, s2_ref, g_ref, b_ref,
             eo_ref, p_ref):
        mu = s1_ref[0:1, :] * (1.0 / _E)
        var = s2_ref[0:1, :] * (1.0 / _E) - mu * mu
        x = en_ref[...]
        xn = g_ref[...] * (x - mu) * lax.rsqrt(var + 1e-5) + b_ref[...]
        ea = jnp.maximum(xn, 0.0)
        sig = jax.nn.sigmoid(ea)
        eo_ref[...] = ee_ref[...] + ea
        p_ref[0] = sig[:, :64]
        p_ref[1] = sig[:, 64:]

    return pl.pallas_call(
        body,
        grid=(_E // _EBLK,),
        in_specs=[
            pl.BlockSpec((_EBLK, _H), lambda i: (i, 0)),
            pl.BlockSpec((_EBLK, _H), lambda i: (i, 0)),
            pl.BlockSpec((8, _H), lambda i: (0, 0)),
            pl.BlockSpec((8, _H), lambda i: (0, 0)),
            pl.BlockSpec((1, _H), lambda i: (0, 0)),
            pl.BlockSpec((1, _H), lambda i: (0, 0)),
        ],
        out_specs=[
            pl.BlockSpec((_EBLK, _H), lambda i: (i, 0)),
            pl.BlockSpec((2, _EBLK, 64), lambda i: (0, i, 0)),
        ],
        out_shape=[
            jax.ShapeDtypeStruct((_E, _H), F32),
            jax.ShapeDtypeStruct((2, _E, 64), F32),
        ],
    )(en, ee, s1, s2, gam, bet)


def _norm_from_deg(degacc):
    def body(d_ref, n_ref):
        deg = d_ref[0, :, 0:1] + d_ref[1, :, 0:1]
        nv = lax.rsqrt(jnp.maximum(deg, 1.0))
        n_ref[...] = jnp.broadcast_to(nv, (_NP, _H))

    return pl.pallas_call(
        body, out_shape=jax.ShapeDtypeStruct((_NP, _H), F32))(degacc)


def _node_a(nm, acc, norm):
    def body(u_ref, a_ref, no_ref, hn_ref, s1_ref, s2_ref):
        ss = jnp.concatenate([a_ref[0, :, :64], a_ref[1, :, :64]], axis=1)
        sh = jnp.concatenate([a_ref[0, :, 64:], a_ref[1, :, 64:]], axis=1)
        hnew = u_ref[0] + no_ref[...] * (sh / (ss + 1e-6))
        hn_ref[...] = hnew

        @pl.when(pl.program_id(0) == 0)
        def _():
            s1_ref[...] = jnp.zeros_like(s1_ref)
            s2_ref[...] = jnp.zeros_like(s2_ref)

        s1_ref[...] += jnp.broadcast_to(
            jnp.sum(hnew, axis=0, keepdims=True), (8, _H))
        s2_ref[...] += jnp.broadcast_to(
            jnp.sum(hnew * hnew, axis=0, keepdims=True), (8, _H))

    return pl.pallas_call(
        body,
        grid=(_N // _NBLK,),
        in_specs=[
            pl.BlockSpec((1, _NBLK, _H), lambda i: (2, i, 0)),
            pl.BlockSpec((2, _NBLK, _H), lambda i: (0, i, 0)),
            pl.BlockSpec((_NBLK, _H), lambda i: (i, 0)),
        ],
        out_specs=[
            pl.BlockSpec((_NBLK, _H), lambda i: (i, 0)),
            pl.BlockSpec((8, _H), lambda i: (0, 0)),
            pl.BlockSpec((8, _H), lambda i: (0, 0)),
        ],
        out_shape=[
            jax.ShapeDtypeStruct((_N, _H), F32),
            jax.ShapeDtypeStruct((8, _H), F32),
            jax.ShapeDtypeStruct((8, _H), F32),
        ],
    )(nm, acc, norm)


def _node_b(hn, hin, s1, s2, gam, bet):
    def body(hn_ref, hi_ref, s1_ref, s2_ref, g_ref, b_ref, ho_ref, hs_ref):
        mu = s1_ref[0:1, :] * (1.0 / _N)
        var = s2_ref[0:1, :] * (1.0 / _N) - mu * mu
        x = hn_ref[...]
        ha = jnp.maximum(
            g_ref[...] * (x - mu) * lax.rsqrt(var + 1e-5) + b_ref[...], 0.0)
        hh = hi_ref[...] + ha
        ho_ref[...] = hh

        @pl.when(pl.program_id(0) == 0)
        def _():
            hs_ref[...] = jnp.zeros_like(hs_ref)

        hs_ref[...] += jnp.broadcast_to(
            jnp.sum(hh, axis=0, keepdims=True), (8, _H))

    return pl.pallas_call(
        body,
        grid=(_N // _NBLK,),
        in_specs=[
            pl.BlockSpec((_NBLK, _H), lambda i: (i, 0)),
            pl.BlockSpec((_NBLK, _H), lambda i: (i, 0)),
            pl.BlockSpec((8, _H), lambda i: (0, 0)),
            pl.BlockSpec((8, _H), lambda i: (0, 0)),
            pl.BlockSpec((1, _H), lambda i: (0, 0)),
            pl.BlockSpec((1, _H), lambda i: (0, 0)),
        ],
        out_specs=[
            pl.BlockSpec((_NBLK, _H), lambda i: (i, 0)),
            pl.BlockSpec((8, _H), lambda i: (0, 0)),
        ],
        out_shape=[
            jax.ShapeDtypeStruct((_N, _H), F32),
            jax.ShapeDtypeStruct((8, _H), F32),
        ],
    )(hn, hin, s1, s2, gam, bet)


def _readout(hsum, w1, b1, w2, b2, w3, b3):
    def body(hs_ref, w1_ref, b1_ref, w2_ref, b2_ref, w3_ref, b3_ref, o_ref):
        hg = hs_ref[...] * (1.0 / _N)
        r = jnp.maximum(
            jnp.dot(hg, w1_ref[...], preferred_element_type=F32)
            + b1_ref[...], 0.0)
        r = jnp.maximum(
            jnp.dot(r, w2_ref[...], preferred_element_type=F32)
            + b2_ref[...], 0.0)
        o_ref[...] = (
            jnp.dot(r, w3_ref[...], preferred_element_type=F32) + b3_ref[...])

    return pl.pallas_call(
        body, out_shape=jax.ShapeDtypeStruct((8, _H), F32))(
            hsum, w1, b1, w2, b2, w3, b3)


def _mesh():
    return plsc.VectorSubcoreMesh(
        core_axis_name="c", subcore_axis_name="s",
        num_cores=2, num_subcores=16)


def _sc_gather_body(ah, bh, src, dst, g_out, idx1, idx2, bufg, sem):
    wid = lax.axis_index("s") * 2 + lax.axis_index("c")
    base = wid * _EW

    def step(i, carry):
        off = base + i * _CH
        pltpu.sync_copy(src.at[pl.ds(off, _CH)], idx1)
        pltpu.sync_copy(dst.at[pl.ds(off, _CH)], idx2)
        pltpu.async_copy(ah.at[idx1], bufg, sem).wait()
        pltpu.async_copy(bh.at[idx2], bufg, sem, add=True).wait()
        pltpu.sync_copy(bufg, g_out.at[pl.ds(off, _CH)])
        return carry

    lax.fori_loop(0, _NCH, step, 0)


def _sc_gather(ah, bh, src, dst):
    return pl.kernel(
        _sc_gather_body,
        out_type=jax.ShapeDtypeStruct((_E, _H), F32),
        mesh=_mesh(),
        scratch_types=[
            pltpu.VMEM((_CH,), jnp.int32),
            pltpu.VMEM((_CH,), jnp.int32),
            pltpu.VMEM((_CH, _H), F32),
            pltpu.SemaphoreType.DMA,
        ],
    )(ah, bh, src, dst)


def _zero_rows(zbuf, acc, s, cols):
    nv = cols // 16

    def zstep(i, carry):
        r = i // nv
        j = i % nv
        zbuf[r, pl.ds(j * 16, 16)] = jnp.zeros((16,), F32)
        return carry

    lax.fori_loop(0, _ZR * nv, zstep, 0)
    for k in range(_NRS // _ZR):
        pltpu.sync_copy(zbuf, acc.at[pl.ds(s * _NRS + k * _ZR, _ZR)])


def _sc_scatter_body(sig, vh, srcidx, dstidx, out, acc,
                     idxs, idxv, bufs, bufv, bufp, zbuf, sem):
    c = lax.axis_index("c")
    s = lax.axis_index("s")
    _zero_rows(zbuf, acc, s, _H)
    plsc.subcore_barrier()
    base = s * _ES
    colbase = c * 64

    def step(i, carry):
        off = base + i * _CH
        pltpu.sync_copy(srcidx.at[pl.ds(off, _CH)], idxs)
        pltpu.sync_copy(dstidx.at[pl.ds(off, _CH)], idxv)
        pltpu.sync_copy(sig.at[c, pl.ds(off, _CH)], bufs)
        pltpu.async_copy(vh.at[idxs], bufv, sem).wait()

        def mstep(r, carry2):
            for j in range(4):
                sl = pl.ds(j * 16, 16)
                sg = bufs[r, sl]
                bufp[r, sl] = sg
                bufp[r, pl.ds(64 + j * 16, 16)] = (
                    sg * bufv[r, pl.ds(colbase + j * 16, 16)])
            return carry2

        lax.fori_loop(0, _CH, mstep, 0)
        pltpu.sync_copy(bufp, acc.at[idxv], add=True)
        return carry

    lax.fori_loop(0, _ES // _CH, step, 0)
    plsc.subcore_barrier()
    stripe = pl.ds(s * _NRS, _NRS)
    pltpu.sync_copy(acc.at[stripe], out.at[c, stripe])


def _sc_scatter(sig, vh, src, dst):
    return pl.kernel(
        _sc_scatter_body,
        out_type=jax.ShapeDtypeStruct((2, _NP, _H), F32),
        mesh=_mesh(),
        scratch_types=[
            pltpu.VMEM_SHARED((_NP, _H), F32),
            pltpu.VMEM((_CH,), jnp.int32),
            pltpu.VMEM((_CH,), jnp.int32),
            pltpu.VMEM((_CH, 64), F32),
            pltpu.VMEM((_CH, _H), F32),
            pltpu.VMEM((_CH, _H), F32),
            pltpu.VMEM((_ZR, _H), F32),
            pltpu.SemaphoreType.DMA,
        ],
    )(sig, vh, src, dst)


def _sc_deg_body(dstidx, out, acc, idxv, onesbuf, zbuf):
    c = lax.axis_index("c")
    s = lax.axis_index("s")
    _zero_rows(zbuf, acc, s, _H)

    def ostep(i, carry):
        r = i // 8
        j = i % 8
        onesbuf[r, pl.ds(j * 16, 16)] = jnp.ones((16,), F32)
        return carry

    lax.fori_loop(0, _CH * 8, ostep, 0)
    plsc.subcore_barrier()
    base = (s * 2 + c) * _EW

    def step(i, carry):
        off = base + i * _CH
        pltpu.sync_copy(dstidx.at[pl.ds(off, _CH)], idxv)
        pltpu.sync_copy(onesbuf, acc.at[idxv], add=True)
        return carry

    lax.fori_loop(0, _NCH, step, 0)
    plsc.subcore_barrier()
    stripe = pl.ds(s * _NRS, _NRS)
    pltpu.sync_copy(acc.at[stripe], out.at[c, stripe])


def _sc_deg(dst):
    return pl.kernel(
        _sc_deg_body,
        out_type=jax.ShapeDtypeStruct((2, _NP, _H), F32),
        mesh=_mesh(),
        scratch_types=[
            pltpu.VMEM_SHARED((_NP, _H), F32),
            pltpu.VMEM((_CH,), jnp.int32),
            pltpu.VMEM((_CH, _H), F32),
            pltpu.VMEM((_ZR, _H), F32),
        ],
    )(dst)


def kernel(h, e, edge_index, Wn, bn, We, be, A, B, C, U, V,
           gh_g, gh_b, ge_g, ge_b, W1, b1, W2, b2, W3, b3):
    src = edge_index[0]
    dst = edge_index[1]

    hh = _mm_bias(h, Wn, bn, _NBLK)
    ee = _mm_bias(e, We, be, _EBLK)

    degacc = _sc_deg(dst)
    norm = _norm_from_deg(degacc)

    for i in range(_L):
        w4 = jnp.stack([A[i], B[i], U[i], V[i]])
        nm = _node_mm(hh, w4)
        g = _sc_gather(nm[0], nm[1], src, dst)
        en, es1, es2 = _edge_a(ee, g, C[i])
        ee, sig = _edge_b(en, ee, es1, es2,
                          ge_g[i].reshape(1, _H), ge_b[i].reshape(1, _H))
        acc = _sc_scatter(sig, nm[3], src, dst)
        hn, ns1, ns2 = _node_a(nm, acc, norm)
        hh, hsum = _node_b(hn, hh, ns1, ns2,
                           gh_g[i].reshape(1, _H), gh_b[i].reshape(1, _H))

    h2 = _H // 2
    h4 = _H // 4
    nc = W3.shape[1]
    w1p = jnp.zeros((_H, _H), F32).at[:, :h2].set(W1)
    b1p = jnp.zeros((1, _H), F32).at[0, :h2].set(b1)
    w2p = jnp.zeros((_H, _H), F32).at[:h2, :h4].set(W2)
    b2p = jnp.zeros((1, _H), F32).at[0, :h4].set(b2)
    w3p = jnp.zeros((_H, _H), F32).at[:h4, :nc].set(W3)
    b3p = jnp.zeros((1, _H), F32).at[0, :nc].set(b3)
    out = _readout(hsum, w1p, b1p, w2p, b2p, w3p, b3p)
    return out[0:1, 0:nc]


# speedup vs baseline: 1.1111x; 1.1111x over previous
import jax
import jax.numpy as jnp
from jax import lax
from jax.experimental import pallas as pl
from jax.experimental.pallas import tpu as pltpu
from jax.experimental.pallas import tpu_sc as plsc

F32 = jnp.float32

_N = 10000
_E = 320000
_E2 = _E // 2
_H = 128
_L = 4

_EBLK = 8000
_E2BLKS = _E2 // _EBLK
_NBLK = 2000
_NW = 32
_NP = 10240
_NRS = _NP // 16
_ZR = 16

_CHG = 40
_EWG = _E2 // _NW
_CHD = 40
_EWD = _E // _NW
_CHS = 80
_ESS = _E2 // 16


def _mm_bias(x, w, b, blk):
    m = x.shape[0]

    def body(x_ref, w_ref, b_ref, o_ref):
        o_ref[...] = (
            jnp.dot(x_ref[...], w_ref[...], preferred_element_type=F32)
            + b_ref[...]
        )

    return pl.pallas_call(
        body,
        grid=(m // blk,),
        in_specs=[
            pl.BlockSpec((blk, _H), lambda i: (i, 0)),
            pl.BlockSpec((_H, _H), lambda i: (0, 0)),
            pl.BlockSpec((1, _H), lambda i: (0, 0)),
        ],
        out_specs=pl.BlockSpec((blk, _H), lambda i: (i, 0)),
        out_shape=jax.ShapeDtypeStruct((m, _H), F32),
    )(x, w, b.reshape(1, _H))


def _node_mm(hh, w4):
    def body(h_ref, w_ref, o_ref):
        for k in range(4):
            o_ref[k] = jnp.dot(h_ref[...], w_ref[k], preferred_element_type=F32)

    return pl.pallas_call(
        body,
        grid=(_N // _NBLK,),
        in_specs=[
            pl.BlockSpec((_NBLK, _H), lambda i: (i, 0)),
            pl.BlockSpec((4, _H, _H), lambda i: (0, 0, 0)),
        ],
        out_specs=pl.BlockSpec((4, _NBLK, _H), lambda i: (0, i, 0)),
        out_shape=jax.ShapeDtypeStruct((4, _N, _H), F32),
    )(hh, w4)


def _edge_a(ee, off, g, c):
    def body(ee_ref, g_ref, c_ref, en_ref, s1_ref, s2_ref):
        en = (
            jnp.dot(ee_ref[...], c_ref[...], preferred_element_type=F32)
            + g_ref[...]
        )
        en_ref[...] = en

        @pl.when(pl.program_id(0) == 0)
        def _():
            s1_ref[...] = jnp.zeros_like(s1_ref)
            s2_ref[...] = jnp.zeros_like(s2_ref)

        s1_ref[...] += jnp.broadcast_to(
            jnp.sum(en, axis=0, keepdims=True), (8, _H))
        s2_ref[...] += jnp.broadcast_to(
            jnp.sum(en * en, axis=0, keepdims=True), (8, _H))

    return pl.pallas_call(
        body,
        grid=(_E2BLKS,),
        in_specs=[
            pl.BlockSpec((_EBLK, _H), lambda i: (i + off, 0)),
            pl.BlockSpec((_EBLK, _H), lambda i: (i, 0)),
            pl.BlockSpec((_H, _H), lambda i: (0, 0)),
        ],
        out_specs=[
            pl.BlockSpec((_EBLK, _H), lambda i: (i, 0)),
            pl.BlockSpec((8, _H), lambda i: (0, 0)),
            pl.BlockSpec((8, _H), lambda i: (0, 0)),
        ],
        out_shape=[
            jax.ShapeDtypeStruct((_E2, _H), F32),
            jax.ShapeDtypeStruct((8, _H), F32),
            jax.ShapeDtypeStruct((8, _H), F32),
        ],
    )(ee, g, c)


def _edge_b(en, ee, off, vs, s1a, s1b, s2a, s2b, gam, bet):
    def body(en_ref, ee_ref, vs_ref, s1a_ref, s1b_ref, s2a_ref, s2b_ref,
             g_ref, b_ref, eo_ref, p_ref):
        mu = (s1a_ref[0:1, :] + s1b_ref[0:1, :]) * (1.0 / _E)
        var = (s2a_ref[0:1, :] + s2b_ref[0:1, :]) * (1.0 / _E) - mu * mu
        x = en_ref[...]
        xn = g_ref[...] * (x - mu) * lax.rsqrt(var + 1e-5) + b_ref[...]
        ea = jnp.maximum(xn, 0.0)
        sig = jax.nn.sigmoid(ea)
        eo_ref[...] = ee_ref[...] + ea
        sv = sig * vs_ref[...]
        p_ref[0] = jnp.concatenate([sig[:, :64], sv[:, :64]], axis=1)
        p_ref[1] = jnp.concatenate([sig[:, 64:], sv[:, 64:]], axis=1)

    stat = pl.BlockSpec((8, _H), lambda i: (0, 0))
    vec = pl.BlockSpec((1, _H), lambda i: (0, 0))
    return pl.pallas_call(
        body,
        grid=(_E2BLKS,),
        in_specs=[
            pl.BlockSpec((_EBLK, _H), lambda i: (i, 0)),
            pl.BlockSpec((_EBLK, _H), lambda i: (i + off, 0)),
            pl.BlockSpec((_EBLK, _H), lambda i: (i, 0)),
            stat, stat, stat, stat, vec, vec,
        ],
        out_specs=[
            pl.BlockSpec((_EBLK, _H), lambda i: (i, 0)),
            pl.BlockSpec((2, _EBLK, _H), lambda i: (0, i, 0)),
        ],
        out_shape=[
            jax.ShapeDtypeStruct((_E2, _H), F32),
            jax.ShapeDtypeStruct((2, _E2, _H), F32),
        ],
    )(en, ee, vs, s1a, s1b, s2a, s2b, gam, bet)


def _norm_from_deg(degacc):
    def body(d_ref, n_ref):
        deg = d_ref[0, :, 0:1] + d_ref[1, :, 0:1]
        nv = lax.rsqrt(jnp.maximum(deg, 1.0))
        n_ref[...] = jnp.broadcast_to(nv, (_NP, _H))

    return pl.pallas_call(
        body, out_shape=jax.ShapeDtypeStruct((_NP, _H), F32))(degacc)


def _node_a(nm, acc0, acc1, norm):
    def body(u_ref, a0_ref, a1_ref, no_ref, hn_ref, s1_ref, s2_ref):
        a = a0_ref[...] + a1_ref[...]
        ss = jnp.concatenate([a[0, :, :64], a[1, :, :64]], axis=1)
        sh = jnp.concatenate([a[0, :, 64:], a[1, :, 64:]], axis=1)
        hnew = u_ref[0] + no_ref[...] * (sh / (ss + 1e-6))
        hn_ref[...] = hnew

        @pl.when(pl.program_id(0) == 0)
        def _():
            s1_ref[...] = jnp.zeros_like(s1_ref)
            s2_ref[...] = jnp.zeros_like(s2_ref)

        s1_ref[...] += jnp.broadcast_to(
            jnp.sum(hnew, axis=0, keepdims=True), (8, _H))
        s2_ref[...] += jnp.broadcast_to(
            jnp.sum(hnew * hnew, axis=0, keepdims=True), (8, _H))

    acc_spec = pl.BlockSpec((2, _NBLK, _H), lambda i: (0, i, 0))
    return pl.pallas_call(
        body,
        grid=(_N // _NBLK,),
        in_specs=[
            pl.BlockSpec((1, _NBLK, _H), lambda i: (2, i, 0)),
            acc_spec,
            acc_spec,
            pl.BlockSpec((_NBLK, _H), lambda i: (i, 0)),
        ],
        out_specs=[
            pl.BlockSpec((_NBLK, _H), lambda i: (i, 0)),
            pl.BlockSpec((8, _H), lambda i: (0, 0)),
            pl.BlockSpec((8, _H), lambda i: (0, 0)),
        ],
        out_shape=[
            jax.ShapeDtypeStruct((_N, _H), F32),
            jax.ShapeDtypeStruct((8, _H), F32),
            jax.ShapeDtypeStruct((8, _H), F32),
        ],
    )(nm, acc0, acc1, norm)


def _node_b(hn, hin, s1, s2, gam, bet):
    def body(hn_ref, hi_ref, s1_ref, s2_ref, g_ref, b_ref, ho_ref, hs_ref):
        mu = s1_ref[0:1, :] * (1.0 / _N)
        var = s2_ref[0:1, :] * (1.0 / _N) - mu * mu
        x = hn_ref[...]
        ha = jnp.maximum(
            g_ref[...] * (x - mu) * lax.rsqrt(var + 1e-5) + b_ref[...], 0.0)
        hh = hi_ref[...] + ha
        ho_ref[...] = hh

        @pl.when(pl.program_id(0) == 0)
        def _():
            hs_ref[...] = jnp.zeros_like(hs_ref)

        hs_ref[...] += jnp.broadcast_to(
            jnp.sum(hh, axis=0, keepdims=True), (8, _H))

    return pl.pallas_call(
        body,
        grid=(_N // _NBLK,),
        in_specs=[
            pl.BlockSpec((_NBLK, _H), lambda i: (i, 0)),
            pl.BlockSpec((_NBLK, _H), lambda i: (i, 0)),
            pl.BlockSpec((8, _H), lambda i: (0, 0)),
            pl.BlockSpec((8, _H), lambda i: (0, 0)),
            pl.BlockSpec((1, _H), lambda i: (0, 0)),
            pl.BlockSpec((1, _H), lambda i: (0, 0)),
        ],
        out_specs=[
            pl.BlockSpec((_NBLK, _H), lambda i: (i, 0)),
            pl.BlockSpec((8, _H), lambda i: (0, 0)),
        ],
        out_shape=[
            jax.ShapeDtypeStruct((_N, _H), F32),
            jax.ShapeDtypeStruct((8, _H), F32),
        ],
    )(hn, hin, s1, s2, gam, bet)


def _readout(hsum, w1, b1, w2, b2, w3, b3):
    def body(hs_ref, w1_ref, b1_ref, w2_ref, b2_ref, w3_ref, b3_ref, o_ref):
        hg = hs_ref[...] * (1.0 / _N)
        r = jnp.maximum(
            jnp.dot(hg, w1_ref[...], preferred_element_type=F32)
            + b1_ref[...], 0.0)
        r = jnp.maximum(
            jnp.dot(r, w2_ref[...], preferred_element_type=F32)
            + b2_ref[...], 0.0)
        o_ref[...] = (
            jnp.dot(r, w3_ref[...], preferred_element_type=F32) + b3_ref[...])

    return pl.pallas_call(
        body, out_shape=jax.ShapeDtypeStruct((8, _H), F32))(
            hsum, w1, b1, w2, b2, w3, b3)


def _mesh():
    return plsc.VectorSubcoreMesh(
        core_axis_name="c", subcore_axis_name="s",
        num_cores=2, num_subcores=16)


def _zero_rows(zbuf, acc, s):
    def zstep(i, carry):
        r = i // 8
        j = i % 8
        zbuf[r, pl.ds(j * 16, 16)] = jnp.zeros((16,), F32)
        return carry

    lax.fori_loop(0, _ZR * 8, zstep, 0)
    for k in range(_NRS // _ZR):
        pltpu.sync_copy(zbuf, acc.at[pl.ds(s * _NRS + k * _ZR, _ZR)])


def _make_gather(half):
    def body(ah, bh, vh, src, dst, g_out, v_out, idx1, idx2, bufg, bufv, sem):
        wid = lax.axis_index("s") * 2 + lax.axis_index("c")
        base_l = wid * _EWG

        def step(i, carry):
            off_l = base_l + i * _CHG
            off_g = half * _E2 + off_l
            pltpu.sync_copy(src.at[pl.ds(off_g, _CHG)], idx1)
            pltpu.sync_copy(dst.at[pl.ds(off_g, _CHG)], idx2)
            pltpu.async_copy(ah.at[idx1], bufg, sem).wait()
            pltpu.async_copy(bh.at[idx2], bufg, sem, add=True).wait()
            pltpu.async_copy(vh.at[idx1], bufv, sem).wait()
            pltpu.sync_copy(bufg, g_out.at[pl.ds(off_l, _CHG)])
            pltpu.sync_copy(bufv, v_out.at[pl.ds(off_l, _CHG)])
            return carry

        lax.fori_loop(0, _EWG // _CHG, step, 0)

    return pl.kernel(
        body,
        out_type=(
            jax.ShapeDtypeStruct((_E2, _H), F32),
            jax.ShapeDtypeStruct((_E2, _H), F32),
        ),
        mesh=_mesh(),
        scratch_types=[
            pltpu.VMEM((_CHG,), jnp.int32),
            pltpu.VMEM((_CHG,), jnp.int32),
            pltpu.VMEM((_CHG, _H), F32),
            pltpu.VMEM((_CHG, _H), F32),
            pltpu.SemaphoreType.DMA,
        ],
    )


def _make_scatter(half):
    def body(pay, dstidx, out, acc, idxv, buf, zbuf):
        c = lax.axis_index("c")
        s = lax.axis_index("s")
        _zero_rows(zbuf, acc, s)
        plsc.subcore_barrier()
        base_l = s * _ESS

        def step(i, carry):
            off_l = base_l + i * _CHS
            off_g = half * _E2 + off_l
            pltpu.sync_copy(dstidx.at[pl.ds(off_g, _CHS)], idxv)
            pltpu.sync_copy(pay.at[c, pl.ds(off_l, _CHS)], buf)
            pltpu.sync_copy(buf, acc.at[idxv], add=True)
            return carry

        lax.fori_loop(0, _ESS // _CHS, step, 0)
        plsc.subcore_barrier()
        stripe = pl.ds(s * _NRS, _NRS)
        pltpu.sync_copy(acc.at[stripe], out.at[c, stripe])

    return pl.kernel(
        body,
        out_type=jax.ShapeDtypeStruct((2, _NP, _H), F32),
        mesh=_mesh(),
        scratch_types=[
            pltpu.VMEM_SHARED((_NP, _H), F32),
            pltpu.VMEM((_CHS,), jnp.int32),
            pltpu.VMEM((_CHS, _H), F32),
            pltpu.VMEM((_ZR, _H), F32),
        ],
    )


def _sc_deg_body(dstidx, out, acc, idxv, onesbuf, zbuf):
    c = lax.axis_index("c")
    s = lax.axis_index("s")
    _zero_rows(zbuf, acc, s)

    def ostep(i, carry):
        r = i // 8
        j = i % 8
        onesbuf[r, pl.ds(j * 16, 16)] = jnp.ones((16,), F32)
        return carry

    lax.fori_loop(0, _CHD * 8, ostep, 0)
    plsc.subcore_barrier()
    base = (s * 2 + c) * _EWD

    def step(i, carry):
        off = base + i * _CHD
        pltpu.sync_copy(dstidx.at[pl.ds(off, _CHD)], idxv)
        pltpu.sync_copy(onesbuf, acc.at[idxv], add=True)
        return carry

    lax.fori_loop(0, _EWD // _CHD, step, 0)
    plsc.subcore_barrier()
    stripe = pl.ds(s * _NRS, _NRS)
    pltpu.sync_copy(acc.at[stripe], out.at[c, stripe])


def _sc_deg(dst):
    return pl.kernel(
        _sc_deg_body,
        out_type=jax.ShapeDtypeStruct((2, _NP, _H), F32),
        mesh=_mesh(),
        scratch_types=[
            pltpu.VMEM_SHARED((_NP, _H), F32),
            pltpu.VMEM((_CHD,), jnp.int32),
            pltpu.VMEM((_CHD, _H), F32),
            pltpu.VMEM((_ZR, _H), F32),
        ],
    )(dst)


def _sc_gather(ah, bh, vh, src, dst, half):
    return _make_gather(half)(ah, bh, vh, src, dst)


def _sc_scatter(pay, dst, half):
    return _make_scatter(half)(pay, dst)


def kernel(h, e, edge_index, Wn, bn, We, be, A, B, C, U, V,
           gh_g, gh_b, ge_g, ge_b, W1, b1, W2, b2, W3, b3):
    src = edge_index[0]
    dst = edge_index[1]

    hh = _mm_bias(h, Wn, bn, _NBLK)
    ee = _mm_bias(e, We, be, _EBLK)

    degacc = _sc_deg(dst)
    norm = _norm_from_deg(degacc)

    ee_h = (ee, ee)
    ee_off = (0, _E2BLKS)
    for i in range(_L):
        w4 = jnp.stack([A[i], B[i], U[i], V[i]])
        nm = _node_mm(hh, w4)
        g0, v0 = _sc_gather(nm[0], nm[1], nm[3], src, dst, 0)
        g1, v1 = _sc_gather(nm[0], nm[1], nm[3], src, dst, 1)
        en0, s10, s20 = _edge_a(ee_h[0], ee_off[0], g0, C[i])
        en1, s11, s21 = _edge_a(ee_h[1], ee_off[1], g1, C[i])
        gam = ge_g[i].reshape(1, _H)
        bet = ge_b[i].reshape(1, _H)
        eo0, pay0 = _edge_b(en0, ee_h[0], ee_off[0], v0,
                            s10, s11, s20, s21, gam, bet)
        acc0 = _sc_scatter(pay0, dst, 0)
        eo1, pay1 = _edge_b(en1, ee_h[1], ee_off[1], v1,
                            s10, s11, s20, s21, gam, bet)
        acc1 = _sc_scatter(pay1, dst, 1)
        ee_h = (eo0, eo1)
        ee_off = (0, 0)
        hn, ns1, ns2 = _node_a(nm, acc0, acc1, norm)
        hh, hsum = _node_b(hn, hh, ns1, ns2,
                           gh_g[i].reshape(1, _H), gh_b[i].reshape(1, _H))

    h2 = _H // 2
    h4 = _H // 4
    nc = W3.shape[1]
    w1p = jnp.zeros((_H, _H), F32).at[:, :h2].set(W1)
    b1p = jnp.zeros((1, _H), F32).at[0, :h2].set(b1)
    w2p = jnp.zeros((_H, _H), F32).at[:h2, :h4].set(W2)
    b2p = jnp.zeros((1, _H), F32).at[0, :h4].set(b2)
    w3p = jnp.zeros((_H, _H), F32).at[:h4, :nc].set(W3)
    b3p = jnp.zeros((1, _H), F32).at[0, :nc].set(b3)
    out = _readout(hsum, w1p, b1p, w2p, b2p, w3p, b3p)
    return out[0:1, 0:nc]


# speedup vs baseline: 1.3957x; 1.2561x over previous
import jax
import jax.numpy as jnp
from jax import lax
from jax.experimental import pallas as pl
from jax.experimental.pallas import tpu as pltpu
from jax.experimental.pallas import tpu_sc as plsc

F32 = jnp.float32

_N = 10000
_E = 320000
_E2 = _E
_H = 128
_L = 4

_EBLK = 8000
_E2BLKS = _E2 // _EBLK
_NBLK = 2000
_NW = 32
_NP = 10240
_NRS = _NP // 16
_ZR = 16

_CHG = 80
_EWG = _E2 // _NW
_CHD = 40
_EWD = _E // _NW
_CHS = 80
_ESS = _E2 // 16


def _mm_bias(x, w, b, blk):
    m = x.shape[0]

    def body(x_ref, w_ref, b_ref, o_ref):
        o_ref[...] = (
            jnp.dot(x_ref[...], w_ref[...], preferred_element_type=F32)
            + b_ref[...]
        )

    return pl.pallas_call(
        body,
        grid=(m // blk,),
        in_specs=[
            pl.BlockSpec((blk, _H), lambda i: (i, 0)),
            pl.BlockSpec((_H, _H), lambda i: (0, 0)),
            pl.BlockSpec((1, _H), lambda i: (0, 0)),
        ],
        out_specs=pl.BlockSpec((blk, _H), lambda i: (i, 0)),
        out_shape=jax.ShapeDtypeStruct((m, _H), F32),
    )(x, w, b.reshape(1, _H))


def _node_mm(hh, w4):
    def body(h_ref, w_ref, o_ref):
        for k in range(4):
            o_ref[k] = jnp.dot(h_ref[...], w_ref[k], preferred_element_type=F32)

    return pl.pallas_call(
        body,
        grid=(_N // _NBLK,),
        in_specs=[
            pl.BlockSpec((_NBLK, _H), lambda i: (i, 0)),
            pl.BlockSpec((4, _H, _H), lambda i: (0, 0, 0)),
        ],
        out_specs=pl.BlockSpec((4, _NBLK, _H), lambda i: (0, i, 0)),
        out_shape=jax.ShapeDtypeStruct((4, _N, _H), F32),
    )(hh, w4)


def _edge_a(ee, off, g, c):
    def body(ee_ref, g_ref, c_ref, en_ref, s1_ref, s2_ref):
        en = (
            jnp.dot(ee_ref[...], c_ref[...], preferred_element_type=F32)
            + g_ref[...]
        )
        en_ref[...] = en

        @pl.when(pl.program_id(0) == 0)
        def _():
            s1_ref[...] = jnp.zeros_like(s1_ref)
            s2_ref[...] = jnp.zeros_like(s2_ref)

        s1_ref[...] += jnp.broadcast_to(
            jnp.sum(en, axis=0, keepdims=True), (8, _H))
        s2_ref[...] += jnp.broadcast_to(
            jnp.sum(en * en, axis=0, keepdims=True), (8, _H))

    return pl.pallas_call(
        body,
        grid=(_E2BLKS,),
        in_specs=[
            pl.BlockSpec((_EBLK, _H), lambda i: (i + off, 0)),
            pl.BlockSpec((_EBLK, _H), lambda i: (i, 0)),
            pl.BlockSpec((_H, _H), lambda i: (0, 0)),
        ],
        out_specs=[
            pl.BlockSpec((_EBLK, _H), lambda i: (i, 0)),
            pl.BlockSpec((8, _H), lambda i: (0, 0)),
            pl.BlockSpec((8, _H), lambda i: (0, 0)),
        ],
        out_shape=[
            jax.ShapeDtypeStruct((_E2, _H), F32),
            jax.ShapeDtypeStruct((8, _H), F32),
            jax.ShapeDtypeStruct((8, _H), F32),
        ],
    )(ee, g, c)


def _edge_b(en, ee, off, vs, s1a, s1b, s2a, s2b, gam, bet):
    def body(en_ref, ee_ref, vs_ref, s1a_ref, s1b_ref, s2a_ref, s2b_ref,
             g_ref, b_ref, eo_ref, p_ref):
        mu = (s1a_ref[0:1, :] + s1b_ref[0:1, :]) * (1.0 / _E)
        var = (s2a_ref[0:1, :] + s2b_ref[0:1, :]) * (1.0 / _E) - mu * mu
        x = en_ref[...]
        xn = g_ref[...] * (x - mu) * lax.rsqrt(var + 1e-5) + b_ref[...]
        ea = jnp.maximum(xn, 0.0)
        sig = jax.nn.sigmoid(ea)
        eo_ref[...] = ee_ref[...] + ea
        sv = sig * vs_ref[...]
        p_ref[0] = jnp.concatenate([sig[:, :64], sv[:, :64]], axis=1)
        p_ref[1] = jnp.concatenate([sig[:, 64:], sv[:, 64:]], axis=1)

    stat = pl.BlockSpec((8, _H), lambda i: (0, 0))
    vec = pl.BlockSpec((1, _H), lambda i: (0, 0))
    return pl.pallas_call(
        body,
        grid=(_E2BLKS,),
        in_specs=[
            pl.BlockSpec((_EBLK, _H), lambda i: (i, 0)),
            pl.BlockSpec((_EBLK, _H), lambda i: (i + off, 0)),
            pl.BlockSpec((_EBLK, _H), lambda i: (i, 0)),
            stat, stat, stat, stat, vec, vec,
        ],
        out_specs=[
            pl.BlockSpec((_EBLK, _H), lambda i: (i, 0)),
            pl.BlockSpec((2, _EBLK, _H), lambda i: (0, i, 0)),
        ],
        out_shape=[
            jax.ShapeDtypeStruct((_E2, _H), F32),
            jax.ShapeDtypeStruct((2, _E2, _H), F32),
        ],
    )(en, ee, vs, s1a, s1b, s2a, s2b, gam, bet)


def _norm_from_deg(degacc):
    def body(d_ref, n_ref):
        deg = d_ref[0, :, 0:1] + d_ref[1, :, 0:1]
        nv = lax.rsqrt(jnp.maximum(deg, 1.0))
        n_ref[...] = jnp.broadcast_to(nv, (_NP, _H))

    return pl.pallas_call(
        body, out_shape=jax.ShapeDtypeStruct((_NP, _H), F32))(degacc)


def _node_a(nm, acc0, norm):
    def body(u_ref, a0_ref, no_ref, hn_ref, s1_ref, s2_ref):
        a = a0_ref[...]
        ss = jnp.concatenate([a[0, :, :64], a[1, :, :64]], axis=1)
        sh = jnp.concatenate([a[0, :, 64:], a[1, :, 64:]], axis=1)
        hnew = u_ref[0] + no_ref[...] * (sh / (ss + 1e-6))
        hn_ref[...] = hnew

        @pl.when(pl.program_id(0) == 0)
        def _():
            s1_ref[...] = jnp.zeros_like(s1_ref)
            s2_ref[...] = jnp.zeros_like(s2_ref)

        s1_ref[...] += jnp.broadcast_to(
            jnp.sum(hnew, axis=0, keepdims=True), (8, _H))
        s2_ref[...] += jnp.broadcast_to(
            jnp.sum(hnew * hnew, axis=0, keepdims=True), (8, _H))

    acc_spec = pl.BlockSpec((2, _NBLK, _H), lambda i: (0, i, 0))
    return pl.pallas_call(
        body,
        grid=(_N // _NBLK,),
        in_specs=[
            pl.BlockSpec((1, _NBLK, _H), lambda i: (2, i, 0)),
            acc_spec,
            pl.BlockSpec((_NBLK, _H), lambda i: (i, 0)),
        ],
        out_specs=[
            pl.BlockSpec((_NBLK, _H), lambda i: (i, 0)),
            pl.BlockSpec((8, _H), lambda i: (0, 0)),
            pl.BlockSpec((8, _H), lambda i: (0, 0)),
        ],
        out_shape=[
            jax.ShapeDtypeStruct((_N, _H), F32),
            jax.ShapeDtypeStruct((8, _H), F32),
            jax.ShapeDtypeStruct((8, _H), F32),
        ],
    )(nm, acc0, norm)


def _node_b(hn, hin, s1, s2, gam, bet):
    def body(hn_ref, hi_ref, s1_ref, s2_ref, g_ref, b_ref, ho_ref, hs_ref):
        mu = s1_ref[0:1, :] * (1.0 / _N)
        var = s2_ref[0:1, :] * (1.0 / _N) - mu * mu
        x = hn_ref[...]
        ha = jnp.maximum(
            g_ref[...] * (x - mu) * lax.rsqrt(var + 1e-5) + b_ref[...], 0.0)
        hh = hi_ref[...] + ha
        ho_ref[...] = hh

        @pl.when(pl.program_id(0) == 0)
        def _():
            hs_ref[...] = jnp.zeros_like(hs_ref)

        hs_ref[...] += jnp.broadcast_to(
            jnp.sum(hh, axis=0, keepdims=True), (8, _H))

    return pl.pallas_call(
        body,
        grid=(_N // _NBLK,),
        in_specs=[
            pl.BlockSpec((_NBLK, _H), lambda i: (i, 0)),
            pl.BlockSpec((_NBLK, _H), lambda i: (i, 0)),
            pl.BlockSpec((8, _H), lambda i: (0, 0)),
            pl.BlockSpec((8, _H), lambda i: (0, 0)),
            pl.BlockSpec((1, _H), lambda i: (0, 0)),
            pl.BlockSpec((1, _H), lambda i: (0, 0)),
        ],
        out_specs=[
            pl.BlockSpec((_NBLK, _H), lambda i: (i, 0)),
            pl.BlockSpec((8, _H), lambda i: (0, 0)),
        ],
        out_shape=[
            jax.ShapeDtypeStruct((_N, _H), F32),
            jax.ShapeDtypeStruct((8, _H), F32),
        ],
    )(hn, hin, s1, s2, gam, bet)


def _readout(hsum, w1, b1, w2, b2, w3, b3):
    def body(hs_ref, w1_ref, b1_ref, w2_ref, b2_ref, w3_ref, b3_ref, o_ref):
        hg = hs_ref[...] * (1.0 / _N)
        r = jnp.maximum(
            jnp.dot(hg, w1_ref[...], preferred_element_type=F32)
            + b1_ref[...], 0.0)
        r = jnp.maximum(
            jnp.dot(r, w2_ref[...], preferred_element_type=F32)
            + b2_ref[...], 0.0)
        o_ref[...] = (
            jnp.dot(r, w3_ref[...], preferred_element_type=F32) + b3_ref[...])

    return pl.pallas_call(
        body, out_shape=jax.ShapeDtypeStruct((8, _H), F32))(
            hsum, w1, b1, w2, b2, w3, b3)


def _mesh():
    return plsc.VectorSubcoreMesh(
        core_axis_name="c", subcore_axis_name="s",
        num_cores=2, num_subcores=16)


def _zero_rows(zbuf, acc, s):
    def zstep(i, carry):
        r = i // 8
        j = i % 8
        zbuf[r, pl.ds(j * 16, 16)] = jnp.zeros((16,), F32)
        return carry

    lax.fori_loop(0, _ZR * 8, zstep, 0)
    for k in range(_NRS // _ZR):
        pltpu.sync_copy(zbuf, acc.at[pl.ds(s * _NRS + k * _ZR, _ZR)])


def _make_gather(half):
    def body(ah, bh, vh, src, dst, g_out, v_out,
             idx1, idx2, bufg, bufv, sem, sem2):
        wid = lax.axis_index("s") * 2 + lax.axis_index("c")
        base_l = wid * _EWG

        def step(i, carry):
            off_l = base_l + i * _CHG
            off_g = half * _E2 + off_l
            pltpu.sync_copy(src.at[pl.ds(off_g, _CHG)], idx1)
            pltpu.sync_copy(dst.at[pl.ds(off_g, _CHG)], idx2)
            ca = pltpu.async_copy(ah.at[idx1], bufg, sem)
            cv = pltpu.async_copy(vh.at[idx1], bufv, sem2)
            ca.wait()
            pltpu.async_copy(bh.at[idx2], bufg, sem, add=True).wait()
            cv.wait()
            pltpu.sync_copy(bufg, g_out.at[pl.ds(off_l, _CHG)])
            pltpu.sync_copy(bufv, v_out.at[pl.ds(off_l, _CHG)])
            return carry

        lax.fori_loop(0, _EWG // _CHG, step, 0)

    return pl.kernel(
        body,
        out_type=(
            jax.ShapeDtypeStruct((_E2, _H), F32),
            jax.ShapeDtypeStruct((_E2, _H), F32),
        ),
        mesh=_mesh(),
        scratch_types=[
            pltpu.VMEM((_CHG,), jnp.int32),
            pltpu.VMEM((_CHG,), jnp.int32),
            pltpu.VMEM((_CHG, _H), F32),
            pltpu.VMEM((_CHG, _H), F32),
            pltpu.SemaphoreType.DMA,
            pltpu.SemaphoreType.DMA,
        ],
    )


def _make_scatter(half):
    def body(pay, dstidx, out, acc, idxv, buf, zbuf):
        c = lax.axis_index("c")
        s = lax.axis_index("s")
        _zero_rows(zbuf, acc, s)
        plsc.subcore_barrier()
        base_l = s * _ESS

        def step(i, carry):
            off_l = base_l + i * _CHS
            off_g = half * _E2 + off_l
            pltpu.sync_copy(dstidx.at[pl.ds(off_g, _CHS)], idxv)
            pltpu.sync_copy(pay.at[c, pl.ds(off_l, _CHS)], buf)
            pltpu.sync_copy(buf, acc.at[idxv], add=True)
            return carry

        lax.fori_loop(0, _ESS // _CHS, step, 0)
        plsc.subcore_barrier()
        stripe = pl.ds(s * _NRS, _NRS)
        pltpu.sync_copy(acc.at[stripe], out.at[c, stripe])

    return pl.kernel(
        body,
        out_type=jax.ShapeDtypeStruct((2, _NP, _H), F32),
        mesh=_mesh(),
        scratch_types=[
            pltpu.VMEM_SHARED((_NP, _H), F32),
            pltpu.VMEM((_CHS,), jnp.int32),
            pltpu.VMEM((_CHS, _H), F32),
            pltpu.VMEM((_ZR, _H), F32),
        ],
    )


def _sc_deg_body(dstidx, out, acc, idxv, onesbuf, zbuf):
    c = lax.axis_index("c")
    s = lax.axis_index("s")
    _zero_rows(zbuf, acc, s)

    def ostep(i, carry):
        r = i // 8
        j = i % 8
        onesbuf[r, pl.ds(j * 16, 16)] = jnp.ones((16,), F32)
        return carry

    lax.fori_loop(0, _CHD * 8, ostep, 0)
    plsc.subcore_barrier()
    base = (s * 2 + c) * _EWD

    def step(i, carry):
        off = base + i * _CHD
        pltpu.sync_copy(dstidx.at[pl.ds(off, _CHD)], idxv)
        pltpu.sync_copy(onesbuf, acc.at[idxv], add=True)
        return carry

    lax.fori_loop(0, _EWD // _CHD, step, 0)
    plsc.subcore_barrier()
    stripe = pl.ds(s * _NRS, _NRS)
    pltpu.sync_copy(acc.at[stripe], out.at[c, stripe])


def _sc_deg(dst):
    return pl.kernel(
        _sc_deg_body,
        out_type=jax.ShapeDtypeStruct((2, _NP, _H), F32),
        mesh=_mesh(),
        scratch_types=[
            pltpu.VMEM_SHARED((_NP, _H), F32),
            pltpu.VMEM((_CHD,), jnp.int32),
            pltpu.VMEM((_CHD, _H), F32),
            pltpu.VMEM((_ZR, _H), F32),
        ],
    )(dst)


def _sc_gather(ah, bh, vh, src, dst, half):
    return _make_gather(half)(ah, bh, vh, src, dst)


def _sc_scatter(pay, dst, half):
    return _make_scatter(half)(pay, dst)


def kernel(h, e, edge_index, Wn, bn, We, be, A, B, C, U, V,
           gh_g, gh_b, ge_g, ge_b, W1, b1, W2, b2, W3, b3):
    src = edge_index[0]
    dst = edge_index[1]

    hh = _mm_bias(h, Wn, bn, _NBLK)
    ee = _mm_bias(e, We, be, _EBLK)

    degacc = _sc_deg(dst)
    norm = _norm_from_deg(degacc)

    zstat = jnp.zeros((8, _H), F32)
    for i in range(_L):
        w4 = jnp.stack([A[i], B[i], U[i], V[i]])
        nm = _node_mm(hh, w4)
        g0, v0 = _sc_gather(nm[0], nm[1], nm[3], src, dst, 0)
        en0, s10, s20 = _edge_a(ee, 0, g0, C[i])
        gam = ge_g[i].reshape(1, _H)
        bet = ge_b[i].reshape(1, _H)
        ee, pay0 = _edge_b(en0, ee, 0, v0,
                           s10, zstat, s20, zstat, gam, bet)
        acc0 = _sc_scatter(pay0, dst, 0)
        hn, ns1, ns2 = _node_a(nm, acc0, norm)
        hh, hsum = _node_b(hn, hh, ns1, ns2,
                           gh_g[i].reshape(1, _H), gh_b[i].reshape(1, _H))

    h2 = _H // 2
    h4 = _H // 4
    nc = W3.shape[1]
    w1p = jnp.zeros((_H, _H), F32).at[:, :h2].set(W1)
    b1p = jnp.zeros((1, _H), F32).at[0, :h2].set(b1)
    w2p = jnp.zeros((_H, _H), F32).at[:h2, :h4].set(W2)
    b2p = jnp.zeros((1, _H), F32).at[0, :h4].set(b2)
    w3p = jnp.zeros((_H, _H), F32).at[:h4, :nc].set(W3)
    b3p = jnp.zeros((1, _H), F32).at[0, :nc].set(b3)
    out = _readout(hsum, w1p, b1p, w2p, b2p, w3p, b3p)
    return out[0:1, 0:nc]


# speedup vs baseline: 1.4961x; 1.0719x over previous
import jax
import jax.numpy as jnp
from jax import lax
from jax.experimental import pallas as pl
from jax.experimental.pallas import tpu as pltpu
from jax.experimental.pallas import tpu_sc as plsc

F32 = jnp.float32

_N = 10000
_E = 320000
_E2 = _E
_H = 128
_L = 4

_EBLK = 8000
_E2BLKS = _E2 // _EBLK
_NBLK = 2000
_NW = 32
_NP = 10240
_NRS = _NP // 16
_ZR = 16

_CHG = 80
_EWG = _E2 // _NW
_CHD = 40
_EWD = _E // _NW
_CHS = 80
_ESS = _E2 // 16


def _mm_bias(x, w, b, blk):
    m = x.shape[0]

    def body(x_ref, w_ref, b_ref, o_ref):
        o_ref[...] = (
            jnp.dot(x_ref[...], w_ref[...], preferred_element_type=F32)
            + b_ref[...]
        )

    return pl.pallas_call(
        body,
        grid=(m // blk,),
        in_specs=[
            pl.BlockSpec((blk, _H), lambda i: (i, 0)),
            pl.BlockSpec((_H, _H), lambda i: (0, 0)),
            pl.BlockSpec((1, _H), lambda i: (0, 0)),
        ],
        out_specs=pl.BlockSpec((blk, _H), lambda i: (i, 0)),
        out_shape=jax.ShapeDtypeStruct((m, _H), F32),
    )(x, w, b.reshape(1, _H))


def _node_mm(hh, w4):
    def body(h_ref, w_ref, o_ref):
        for k in range(4):
            o_ref[k] = jnp.dot(h_ref[...], w_ref[k], preferred_element_type=F32)

    return pl.pallas_call(
        body,
        grid=(_N // _NBLK,),
        in_specs=[
            pl.BlockSpec((_NBLK, _H), lambda i: (i, 0)),
            pl.BlockSpec((4, _H, _H), lambda i: (0, 0, 0)),
        ],
        out_specs=pl.BlockSpec((4, _NBLK, _H), lambda i: (0, i, 0)),
        out_shape=jax.ShapeDtypeStruct((4, _N, _H), F32),
    )(hh, w4)


def _edge_a(ee, off, g, c):
    def body(ee_ref, g_ref, c_ref, en_ref, s1_ref, s2_ref):
        en = (
            jnp.dot(ee_ref[...], c_ref[...], preferred_element_type=F32)
            + g_ref[...]
        )
        en_ref[...] = en

        @pl.when(pl.program_id(0) == 0)
        def _():
            s1_ref[...] = jnp.zeros_like(s1_ref)
            s2_ref[...] = jnp.zeros_like(s2_ref)

        s1_ref[...] += jnp.broadcast_to(
            jnp.sum(en, axis=0, keepdims=True), (8, _H))
        s2_ref[...] += jnp.broadcast_to(
            jnp.sum(en * en, axis=0, keepdims=True), (8, _H))

    return pl.pallas_call(
        body,
        grid=(_E2BLKS,),
        in_specs=[
            pl.BlockSpec((_EBLK, _H), lambda i: (i + off, 0)),
            pl.BlockSpec((_EBLK, _H), lambda i: (i, 0)),
            pl.BlockSpec((_H, _H), lambda i: (0, 0)),
        ],
        out_specs=[
            pl.BlockSpec((_EBLK, _H), lambda i: (i, 0)),
            pl.BlockSpec((8, _H), lambda i: (0, 0)),
            pl.BlockSpec((8, _H), lambda i: (0, 0)),
        ],
        out_shape=[
            jax.ShapeDtypeStruct((_E2, _H), F32),
            jax.ShapeDtypeStruct((8, _H), F32),
            jax.ShapeDtypeStruct((8, _H), F32),
        ],
    )(ee, g, c)


def _edge_b(en, ee, off, vs, s1a, s1b, s2a, s2b, gam, bet):
    def body(en_ref, ee_ref, vs_ref, s1a_ref, s1b_ref, s2a_ref, s2b_ref,
             g_ref, b_ref, eo_ref, p_ref):
        mu = (s1a_ref[0:1, :] + s1b_ref[0:1, :]) * (1.0 / _E)
        var = (s2a_ref[0:1, :] + s2b_ref[0:1, :]) * (1.0 / _E) - mu * mu
        x = en_ref[...]
        xn = g_ref[...] * (x - mu) * lax.rsqrt(var + 1e-5) + b_ref[...]
        ea = jnp.maximum(xn, 0.0)
        sig = jax.nn.sigmoid(ea)
        eo_ref[...] = ee_ref[...] + ea
        sv = sig * vs_ref[...]
        p_ref[0] = jnp.concatenate([sig[:, :64], sv[:, :64]], axis=1)
        p_ref[1] = jnp.concatenate([sig[:, 64:], sv[:, 64:]], axis=1)

    stat = pl.BlockSpec((8, _H), lambda i: (0, 0))
    vec = pl.BlockSpec((1, _H), lambda i: (0, 0))
    return pl.pallas_call(
        body,
        grid=(_E2BLKS,),
        in_specs=[
            pl.BlockSpec((_EBLK, _H), lambda i: (i, 0)),
            pl.BlockSpec((_EBLK, _H), lambda i: (i + off, 0)),
            pl.BlockSpec((_EBLK, _H), lambda i: (i, 0)),
            stat, stat, stat, stat, vec, vec,
        ],
        out_specs=[
            pl.BlockSpec((_EBLK, _H), lambda i: (i, 0)),
            pl.BlockSpec((2, _EBLK, _H), lambda i: (0, i, 0)),
        ],
        out_shape=[
            jax.ShapeDtypeStruct((_E2, _H), F32),
            jax.ShapeDtypeStruct((2, _E2, _H), F32),
        ],
    )(en, ee, vs, s1a, s1b, s2a, s2b, gam, bet)


def _norm_from_deg(degacc):
    def body(d_ref, n_ref):
        deg = d_ref[0, :, 0:1] + d_ref[1, :, 0:1]
        nv = lax.rsqrt(jnp.maximum(deg, 1.0))
        n_ref[...] = jnp.broadcast_to(nv, (_NP, _H))

    return pl.pallas_call(
        body, out_shape=jax.ShapeDtypeStruct((_NP, _H), F32))(degacc)


def _node_a(nm, acc0, norm):
    def body(u_ref, a0_ref, no_ref, hn_ref, s1_ref, s2_ref):
        a = a0_ref[...]
        ss = jnp.concatenate([a[0, :, :64], a[1, :, :64]], axis=1)
        sh = jnp.concatenate([a[0, :, 64:], a[1, :, 64:]], axis=1)
        hnew = u_ref[0] + no_ref[...] * (sh / (ss + 1e-6))
        hn_ref[...] = hnew

        @pl.when(pl.program_id(0) == 0)
        def _():
            s1_ref[...] = jnp.zeros_like(s1_ref)
            s2_ref[...] = jnp.zeros_like(s2_ref)

        s1_ref[...] += jnp.broadcast_to(
            jnp.sum(hnew, axis=0, keepdims=True), (8, _H))
        s2_ref[...] += jnp.broadcast_to(
            jnp.sum(hnew * hnew, axis=0, keepdims=True), (8, _H))

    acc_spec = pl.BlockSpec((2, _NBLK, _H), lambda i: (0, i, 0))
    return pl.pallas_call(
        body,
        grid=(_N // _NBLK,),
        in_specs=[
            pl.BlockSpec((1, _NBLK, _H), lambda i: (2, i, 0)),
            acc_spec,
            pl.BlockSpec((_NBLK, _H), lambda i: (i, 0)),
        ],
        out_specs=[
            pl.BlockSpec((_NBLK, _H), lambda i: (i, 0)),
            pl.BlockSpec((8, _H), lambda i: (0, 0)),
            pl.BlockSpec((8, _H), lambda i: (0, 0)),
        ],
        out_shape=[
            jax.ShapeDtypeStruct((_N, _H), F32),
            jax.ShapeDtypeStruct((8, _H), F32),
            jax.ShapeDtypeStruct((8, _H), F32),
        ],
    )(nm, acc0, norm)


def _node_b(hn, hin, s1, s2, gam, bet):
    def body(hn_ref, hi_ref, s1_ref, s2_ref, g_ref, b_ref, ho_ref, hs_ref):
        mu = s1_ref[0:1, :] * (1.0 / _N)
        var = s2_ref[0:1, :] * (1.0 / _N) - mu * mu
        x = hn_ref[...]
        ha = jnp.maximum(
            g_ref[...] * (x - mu) * lax.rsqrt(var + 1e-5) + b_ref[...], 0.0)
        hh = hi_ref[...] + ha
        ho_ref[...] = hh

        @pl.when(pl.program_id(0) == 0)
        def _():
            hs_ref[...] = jnp.zeros_like(hs_ref)

        hs_ref[...] += jnp.broadcast_to(
            jnp.sum(hh, axis=0, keepdims=True), (8, _H))

    return pl.pallas_call(
        body,
        grid=(_N // _NBLK,),
        in_specs=[
            pl.BlockSpec((_NBLK, _H), lambda i: (i, 0)),
            pl.BlockSpec((_NBLK, _H), lambda i: (i, 0)),
            pl.BlockSpec((8, _H), lambda i: (0, 0)),
            pl.BlockSpec((8, _H), lambda i: (0, 0)),
            pl.BlockSpec((1, _H), lambda i: (0, 0)),
            pl.BlockSpec((1, _H), lambda i: (0, 0)),
        ],
        out_specs=[
            pl.BlockSpec((_NBLK, _H), lambda i: (i, 0)),
            pl.BlockSpec((8, _H), lambda i: (0, 0)),
        ],
        out_shape=[
            jax.ShapeDtypeStruct((_N, _H), F32),
            jax.ShapeDtypeStruct((8, _H), F32),
        ],
    )(hn, hin, s1, s2, gam, bet)


def _readout(hsum, w1, b1, w2, b2, w3, b3):
    def body(hs_ref, w1_ref, b1_ref, w2_ref, b2_ref, w3_ref, b3_ref, o_ref):
        hg = hs_ref[...] * (1.0 / _N)
        r = jnp.maximum(
            jnp.dot(hg, w1_ref[...], preferred_element_type=F32)
            + b1_ref[...], 0.0)
        r = jnp.maximum(
            jnp.dot(r, w2_ref[...], preferred_element_type=F32)
            + b2_ref[...], 0.0)
        o_ref[...] = (
            jnp.dot(r, w3_ref[...], preferred_element_type=F32) + b3_ref[...])

    return pl.pallas_call(
        body, out_shape=jax.ShapeDtypeStruct((8, _H), F32))(
            hsum, w1, b1, w2, b2, w3, b3)


def _mesh():
    return plsc.VectorSubcoreMesh(
        core_axis_name="c", subcore_axis_name="s",
        num_cores=2, num_subcores=16)


def _zero_rows(zbuf, acc, s):
    def zstep(i, carry):
        r = i // 8
        j = i % 8
        zbuf[r, pl.ds(j * 16, 16)] = jnp.zeros((16,), F32)
        return carry

    lax.fori_loop(0, _ZR * 8, zstep, 0)
    for k in range(_NRS // _ZR):
        pltpu.sync_copy(zbuf, acc.at[pl.ds(s * _NRS + k * _ZR, _ZR)])


def _make_gather(half):
    def body(ah, bh, vh, src, dst, g_out, v_out,
             idx1, idx2, bufg, bufv, sem, sem2):
        wid = lax.axis_index("s") * 2 + lax.axis_index("c")
        base_l = wid * _EWG

        def step(i, carry):
            off_l = base_l + i * _CHG
            off_g = half * _E2 + off_l
            pltpu.sync_copy(src.at[pl.ds(off_g, _CHG)], idx1)
            pltpu.sync_copy(dst.at[pl.ds(off_g, _CHG)], idx2)
            ca = pltpu.async_copy(ah.at[idx1], bufg, sem)
            cv = pltpu.async_copy(vh.at[idx1], bufv, sem2)
            ca.wait()
            pltpu.async_copy(bh.at[idx2], bufg, sem, add=True).wait()
            cv.wait()
            pltpu.sync_copy(bufg, g_out.at[pl.ds(off_l, _CHG)])
            pltpu.sync_copy(bufv, v_out.at[pl.ds(off_l, _CHG)])
            return carry

        lax.fori_loop(0, _EWG // _CHG, step, 0)

    return pl.kernel(
        body,
        out_type=(
            jax.ShapeDtypeStruct((_E2, _H), F32),
            jax.ShapeDtypeStruct((_E2, _H), F32),
        ),
        mesh=_mesh(),
        scratch_types=[
            pltpu.VMEM((_CHG,), jnp.int32),
            pltpu.VMEM((_CHG,), jnp.int32),
            pltpu.VMEM((_CHG, _H), F32),
            pltpu.VMEM((_CHG, _H), F32),
            pltpu.SemaphoreType.DMA,
            pltpu.SemaphoreType.DMA,
        ],
    )


def _make_scatter(half):
    def body(pay, dstidx, out, acc, idxv, buf, zbuf, sem, sem2):
        c = lax.axis_index("c")
        s = lax.axis_index("s")
        _zero_rows(zbuf, acc, s)
        plsc.subcore_barrier()
        base_l = s * _ESS

        def step(i, carry):
            off_l = base_l + i * _CHS
            off_g = half * _E2 + off_l
            ci = pltpu.async_copy(dstidx.at[pl.ds(off_g, _CHS)], idxv, sem)
            cp = pltpu.async_copy(pay.at[c, pl.ds(off_l, _CHS)], buf, sem2)
            ci.wait()
            cp.wait()
            pltpu.sync_copy(buf, acc.at[idxv], add=True)
            return carry

        lax.fori_loop(0, _ESS // _CHS, step, 0)
        plsc.subcore_barrier()
        stripe = pl.ds(s * _NRS, _NRS)
        pltpu.sync_copy(acc.at[stripe], out.at[c, stripe])

    return pl.kernel(
        body,
        out_type=jax.ShapeDtypeStruct((2, _NP, _H), F32),
        mesh=_mesh(),
        scratch_types=[
            pltpu.VMEM_SHARED((_NP, _H), F32),
            pltpu.VMEM((_CHS,), jnp.int32),
            pltpu.VMEM((_CHS, _H), F32),
            pltpu.VMEM((_ZR, _H), F32),
            pltpu.SemaphoreType.DMA,
            pltpu.SemaphoreType.DMA,
        ],
    )


def _sc_deg_body(dstidx, out, acc, idxv, onesbuf, zbuf):
    c = lax.axis_index("c")
    s = lax.axis_index("s")
    _zero_rows(zbuf, acc, s)

    def ostep(i, carry):
        r = i // 8
        j = i % 8
        onesbuf[r, pl.ds(j * 16, 16)] = jnp.ones((16,), F32)
        return carry

    lax.fori_loop(0, _CHD * 8, ostep, 0)
    plsc.subcore_barrier()
    base = (s * 2 + c) * _EWD

    def step(i, carry):
        off = base + i * _CHD
        pltpu.sync_copy(dstidx.at[pl.ds(off, _CHD)], idxv)
        pltpu.sync_copy(onesbuf, acc.at[idxv], add=True)
        return carry

    lax.fori_loop(0, _EWD // _CHD, step, 0)
    plsc.subcore_barrier()
    stripe = pl.ds(s * _NRS, _NRS)
    pltpu.sync_copy(acc.at[stripe], out.at[c, stripe])


def _sc_deg(dst):
    return pl.kernel(
        _sc_deg_body,
        out_type=jax.ShapeDtypeStruct((2, _NP, _H), F32),
        mesh=_mesh(),
        scratch_types=[
            pltpu.VMEM_SHARED((_NP, _H), F32),
            pltpu.VMEM((_CHD,), jnp.int32),
            pltpu.VMEM((_CHD, _H), F32),
            pltpu.VMEM((_ZR, _H), F32),
        ],
    )(dst)


def _sc_gather(ah, bh, vh, src, dst, half):
    return _make_gather(half)(ah, bh, vh, src, dst)


def _sc_scatter(pay, dst, half):
    return _make_scatter(half)(pay, dst)


def kernel(h, e, edge_index, Wn, bn, We, be, A, B, C, U, V,
           gh_g, gh_b, ge_g, ge_b, W1, b1, W2, b2, W3, b3):
    src = edge_index[0]
    dst = edge_index[1]

    hh = _mm_bias(h, Wn, bn, _NBLK)
    ee = _mm_bias(e, We, be, _EBLK)

    degacc = _sc_deg(dst)
    norm = _norm_from_deg(degacc)

    zstat = jnp.zeros((8, _H), F32)
    for i in range(_L):
        w4 = jnp.stack([A[i], B[i], U[i], V[i]])
        nm = _node_mm(hh, w4)
        g0, v0 = _sc_gather(nm[0], nm[1], nm[3], src, dst, 0)
        en0, s10, s20 = _edge_a(ee, 0, g0, C[i])
        gam = ge_g[i].reshape(1, _H)
        bet = ge_b[i].reshape(1, _H)
        ee, pay0 = _edge_b(en0, ee, 0, v0,
                           s10, zstat, s20, zstat, gam, bet)
        acc0 = _sc_scatter(pay0, dst, 0)
        hn, ns1, ns2 = _node_a(nm, acc0, norm)
        hh, hsum = _node_b(hn, hh, ns1, ns2,
                           gh_g[i].reshape(1, _H), gh_b[i].reshape(1, _H))

    h2 = _H // 2
    h4 = _H // 4
    nc = W3.shape[1]
    w1p = jnp.zeros((_H, _H), F32).at[:, :h2].set(W1)
    b1p = jnp.zeros((1, _H), F32).at[0, :h2].set(b1)
    w2p = jnp.zeros((_H, _H), F32).at[:h2, :h4].set(W2)
    b2p = jnp.zeros((1, _H), F32).at[0, :h4].set(b2)
    w3p = jnp.zeros((_H, _H), F32).at[:h4, :nc].set(W3)
    b3p = jnp.zeros((1, _H), F32).at[0, :nc].set(b3)
    out = _readout(hsum, w1p, b1p, w2p, b2p, w3p, b3p)
    return out[0:1, 0:nc]


# speedup vs baseline: 1.5573x; 1.0409x over previous
import jax
import jax.numpy as jnp
from jax import lax
from jax.experimental import pallas as pl
from jax.experimental.pallas import tpu as pltpu
from jax.experimental.pallas import tpu_sc as plsc

F32 = jnp.float32

_N = 10000
_E = 320000
_E2 = _E
_H = 128
_L = 4

_EBLK = 8000
_E2BLKS = _E2 // _EBLK
_NBLK = 2000
_NW = 32
_NP = 10240
_NRS = _NP // 16
_ZR = 16

_CHG = 80
_EWG = _E2 // _NW
_CHD = 40
_EWD = _E // _NW
_CHS = 80
_ESS = _E2 // 16


def _mm_bias(x, w, b, blk):
    m = x.shape[0]

    def body(x_ref, w_ref, b_ref, o_ref):
        o_ref[...] = (
            jnp.dot(x_ref[...], w_ref[...], preferred_element_type=F32)
            + b_ref[...]
        )

    return pl.pallas_call(
        body,
        grid=(m // blk,),
        in_specs=[
            pl.BlockSpec((blk, _H), lambda i: (i, 0)),
            pl.BlockSpec((_H, _H), lambda i: (0, 0)),
            pl.BlockSpec((1, _H), lambda i: (0, 0)),
        ],
        out_specs=pl.BlockSpec((blk, _H), lambda i: (i, 0)),
        out_shape=jax.ShapeDtypeStruct((m, _H), F32),
    )(x, w, b.reshape(1, _H))


def _node_mm(hh, w4):
    def body(h_ref, w_ref, o_ref):
        for k in range(4):
            o_ref[k] = jnp.dot(h_ref[...], w_ref[k], preferred_element_type=F32)

    return pl.pallas_call(
        body,
        grid=(_N // _NBLK,),
        in_specs=[
            pl.BlockSpec((_NBLK, _H), lambda i: (i, 0)),
            pl.BlockSpec((4, _H, _H), lambda i: (0, 0, 0)),
        ],
        out_specs=pl.BlockSpec((4, _NBLK, _H), lambda i: (0, i, 0)),
        out_shape=jax.ShapeDtypeStruct((4, _N, _H), F32),
    )(hh, w4)


def _edge_a(ee, off, g, c):
    def body(ee_ref, g_ref, c_ref, en_ref, s1_ref, s2_ref):
        en = (
            jnp.dot(ee_ref[...], c_ref[...], preferred_element_type=F32)
            + g_ref[...]
        )
        en_ref[...] = en

        @pl.when(pl.program_id(0) == 0)
        def _():
            s1_ref[...] = jnp.zeros_like(s1_ref)
            s2_ref[...] = jnp.zeros_like(s2_ref)

        s1_ref[...] += jnp.broadcast_to(
            jnp.sum(en, axis=0, keepdims=True), (8, _H))
        s2_ref[...] += jnp.broadcast_to(
            jnp.sum(en * en, axis=0, keepdims=True), (8, _H))

    return pl.pallas_call(
        body,
        grid=(_E2BLKS,),
        in_specs=[
            pl.BlockSpec((_EBLK, _H), lambda i: (i + off, 0)),
            pl.BlockSpec((_EBLK, _H), lambda i: (i, 0)),
            pl.BlockSpec((_H, _H), lambda i: (0, 0)),
        ],
        out_specs=[
            pl.BlockSpec((_EBLK, _H), lambda i: (i, 0)),
            pl.BlockSpec((8, _H), lambda i: (0, 0)),
            pl.BlockSpec((8, _H), lambda i: (0, 0)),
        ],
        out_shape=[
            jax.ShapeDtypeStruct((_E2, _H), F32),
            jax.ShapeDtypeStruct((8, _H), F32),
            jax.ShapeDtypeStruct((8, _H), F32),
        ],
    )(ee, g, c)


def _edge_b(en, ee, off, vs, s1a, s1b, s2a, s2b, gam, bet):
    def body(en_ref, ee_ref, vs_ref, s1a_ref, s1b_ref, s2a_ref, s2b_ref,
             g_ref, b_ref, eo_ref, p_ref):
        mu = (s1a_ref[0:1, :] + s1b_ref[0:1, :]) * (1.0 / _E)
        var = (s2a_ref[0:1, :] + s2b_ref[0:1, :]) * (1.0 / _E) - mu * mu
        x = en_ref[...]
        xn = g_ref[...] * (x - mu) * lax.rsqrt(var + 1e-5) + b_ref[...]
        ea = jnp.maximum(xn, 0.0)
        sig = jax.nn.sigmoid(ea)
        eo_ref[...] = ee_ref[...] + ea
        sv = sig * vs_ref[...]
        p_ref[0] = jnp.concatenate([sig[:, :64], sv[:, :64]], axis=1)
        p_ref[1] = jnp.concatenate([sig[:, 64:], sv[:, 64:]], axis=1)

    stat = pl.BlockSpec((8, _H), lambda i: (0, 0))
    vec = pl.BlockSpec((1, _H), lambda i: (0, 0))
    return pl.pallas_call(
        body,
        grid=(_E2BLKS,),
        in_specs=[
            pl.BlockSpec((_EBLK, _H), lambda i: (i, 0)),
            pl.BlockSpec((_EBLK, _H), lambda i: (i + off, 0)),
            pl.BlockSpec((_EBLK, _H), lambda i: (i, 0)),
            stat, stat, stat, stat, vec, vec,
        ],
        out_specs=[
            pl.BlockSpec((_EBLK, _H), lambda i: (i, 0)),
            pl.BlockSpec((2, _EBLK, _H), lambda i: (0, i, 0)),
        ],
        out_shape=[
            jax.ShapeDtypeStruct((_E2, _H), F32),
            jax.ShapeDtypeStruct((2, _E2, _H), F32),
        ],
    )(en, ee, vs, s1a, s1b, s2a, s2b, gam, bet)


def _norm_from_deg(degacc):
    def body(d_ref, n_ref):
        deg = d_ref[0, :, 0:1] + d_ref[1, :, 0:1]
        nv = lax.rsqrt(jnp.maximum(deg, 1.0))
        n_ref[...] = jnp.broadcast_to(nv, (_NP, _H))

    return pl.pallas_call(
        body, out_shape=jax.ShapeDtypeStruct((_NP, _H), F32))(degacc)


def _node_a(nm, acc0, norm):
    def body(u_ref, a0_ref, no_ref, hn_ref, s1_ref, s2_ref):
        a = a0_ref[...]
        ss = jnp.concatenate([a[0, :, :64], a[1, :, :64]], axis=1)
        sh = jnp.concatenate([a[0, :, 64:], a[1, :, 64:]], axis=1)
        hnew = u_ref[0] + no_ref[...] * (sh / (ss + 1e-6))
        hn_ref[...] = hnew

        @pl.when(pl.program_id(0) == 0)
        def _():
            s1_ref[...] = jnp.zeros_like(s1_ref)
            s2_ref[...] = jnp.zeros_like(s2_ref)

        s1_ref[...] += jnp.broadcast_to(
            jnp.sum(hnew, axis=0, keepdims=True), (8, _H))
        s2_ref[...] += jnp.broadcast_to(
            jnp.sum(hnew * hnew, axis=0, keepdims=True), (8, _H))

    acc_spec = pl.BlockSpec((2, _NBLK, _H), lambda i: (0, i, 0))
    return pl.pallas_call(
        body,
        grid=(_N // _NBLK,),
        in_specs=[
            pl.BlockSpec((1, _NBLK, _H), lambda i: (2, i, 0)),
            acc_spec,
            pl.BlockSpec((_NBLK, _H), lambda i: (i, 0)),
        ],
        out_specs=[
            pl.BlockSpec((_NBLK, _H), lambda i: (i, 0)),
            pl.BlockSpec((8, _H), lambda i: (0, 0)),
            pl.BlockSpec((8, _H), lambda i: (0, 0)),
        ],
        out_shape=[
            jax.ShapeDtypeStruct((_N, _H), F32),
            jax.ShapeDtypeStruct((8, _H), F32),
            jax.ShapeDtypeStruct((8, _H), F32),
        ],
    )(nm, acc0, norm)


def _node_b(hn, hin, s1, s2, gam, bet):
    def body(hn_ref, hi_ref, s1_ref, s2_ref, g_ref, b_ref, ho_ref, hs_ref):
        mu = s1_ref[0:1, :] * (1.0 / _N)
        var = s2_ref[0:1, :] * (1.0 / _N) - mu * mu
        x = hn_ref[...]
        ha = jnp.maximum(
            g_ref[...] * (x - mu) * lax.rsqrt(var + 1e-5) + b_ref[...], 0.0)
        hh = hi_ref[...] + ha
        ho_ref[...] = hh

        @pl.when(pl.program_id(0) == 0)
        def _():
            hs_ref[...] = jnp.zeros_like(hs_ref)

        hs_ref[...] += jnp.broadcast_to(
            jnp.sum(hh, axis=0, keepdims=True), (8, _H))

    return pl.pallas_call(
        body,
        grid=(_N // _NBLK,),
        in_specs=[
            pl.BlockSpec((_NBLK, _H), lambda i: (i, 0)),
            pl.BlockSpec((_NBLK, _H), lambda i: (i, 0)),
            pl.BlockSpec((8, _H), lambda i: (0, 0)),
            pl.BlockSpec((8, _H), lambda i: (0, 0)),
            pl.BlockSpec((1, _H), lambda i: (0, 0)),
            pl.BlockSpec((1, _H), lambda i: (0, 0)),
        ],
        out_specs=[
            pl.BlockSpec((_NBLK, _H), lambda i: (i, 0)),
            pl.BlockSpec((8, _H), lambda i: (0, 0)),
        ],
        out_shape=[
            jax.ShapeDtypeStruct((_N, _H), F32),
            jax.ShapeDtypeStruct((8, _H), F32),
        ],
    )(hn, hin, s1, s2, gam, bet)


def _readout(hsum, w1, b1, w2, b2, w3, b3):
    def body(hs_ref, w1_ref, b1_ref, w2_ref, b2_ref, w3_ref, b3_ref, o_ref):
        hg = hs_ref[...] * (1.0 / _N)
        r = jnp.maximum(
            jnp.dot(hg, w1_ref[...], preferred_element_type=F32)
            + b1_ref[...], 0.0)
        r = jnp.maximum(
            jnp.dot(r, w2_ref[...], preferred_element_type=F32)
            + b2_ref[...], 0.0)
        o_ref[...] = (
            jnp.dot(r, w3_ref[...], preferred_element_type=F32) + b3_ref[...])

    return pl.pallas_call(
        body, out_shape=jax.ShapeDtypeStruct((8, _H), F32))(
            hsum, w1, b1, w2, b2, w3, b3)


def _mesh():
    return plsc.VectorSubcoreMesh(
        core_axis_name="c", subcore_axis_name="s",
        num_cores=2, num_subcores=16)


def _zero_rows(zbuf, acc, s):
    def zstep(i, carry):
        r = i // 8
        j = i % 8
        zbuf[r, pl.ds(j * 16, 16)] = jnp.zeros((16,), F32)
        return carry

    lax.fori_loop(0, _ZR * 8, zstep, 0)
    for k in range(_NRS // _ZR):
        pltpu.sync_copy(zbuf, acc.at[pl.ds(s * _NRS + k * _ZR, _ZR)])


def _make_gather(half):
    def body(ah, bh, vh, src, dst, g_out, v_out,
             idx1, idx2, bufg, bufv, sem, sem2):
        wid = lax.axis_index("s") * 2 + lax.axis_index("c")
        base_l = wid * _EWG

        def step(i, carry):
            off_l = base_l + i * _CHG
            off_g = half * _E2 + off_l
            c1 = pltpu.async_copy(src.at[pl.ds(off_g, _CHG)], idx1, sem)
            c2 = pltpu.async_copy(dst.at[pl.ds(off_g, _CHG)], idx2, sem2)
            c1.wait()
            c2.wait()
            ca = pltpu.async_copy(ah.at[idx1], bufg, sem)
            cv = pltpu.async_copy(vh.at[idx1], bufv, sem2)
            ca.wait()
            pltpu.async_copy(bh.at[idx2], bufg, sem, add=True).wait()
            cv.wait()
            cg = pltpu.async_copy(bufg, g_out.at[pl.ds(off_l, _CHG)], sem)
            cw = pltpu.async_copy(bufv, v_out.at[pl.ds(off_l, _CHG)], sem2)
            cg.wait()
            cw.wait()
            return carry

        lax.fori_loop(0, _EWG // _CHG, step, 0)

    return pl.kernel(
        body,
        out_type=(
            jax.ShapeDtypeStruct((_E2, _H), F32),
            jax.ShapeDtypeStruct((_E2, _H), F32),
        ),
        mesh=_mesh(),
        scratch_types=[
            pltpu.VMEM((_CHG,), jnp.int32),
            pltpu.VMEM((_CHG,), jnp.int32),
            pltpu.VMEM((_CHG, _H), F32),
            pltpu.VMEM((_CHG, _H), F32),
            pltpu.SemaphoreType.DMA,
            pltpu.SemaphoreType.DMA,
        ],
    )


def _make_scatter(half):
    def body(pay, dstidx, out, acc, idxv, buf, zbuf, sem, sem2):
        c = lax.axis_index("c")
        s = lax.axis_index("s")
        _zero_rows(zbuf, acc, s)
        plsc.subcore_barrier()
        base_l = s * _ESS

        def step(i, carry):
            off_l = base_l + i * _CHS
            off_g = half * _E2 + off_l
            ci = pltpu.async_copy(dstidx.at[pl.ds(off_g, _CHS)], idxv, sem)
            cp = pltpu.async_copy(pay.at[c, pl.ds(off_l, _CHS)], buf, sem2)
            ci.wait()
            cp.wait()
            pltpu.sync_copy(buf, acc.at[idxv], add=True)
            return carry

        lax.fori_loop(0, _ESS // _CHS, step, 0)
        plsc.subcore_barrier()
        stripe = pl.ds(s * _NRS, _NRS)
        pltpu.sync_copy(acc.at[stripe], out.at[c, stripe])

    return pl.kernel(
        body,
        out_type=jax.ShapeDtypeStruct((2, _NP, _H), F32),
        mesh=_mesh(),
        scratch_types=[
            pltpu.VMEM_SHARED((_NP, _H), F32),
            pltpu.VMEM((_CHS,), jnp.int32),
            pltpu.VMEM((_CHS, _H), F32),
            pltpu.VMEM((_ZR, _H), F32),
            pltpu.SemaphoreType.DMA,
            pltpu.SemaphoreType.DMA,
        ],
    )


def _sc_deg_body(dstidx, out, acc, idxv, onesbuf, zbuf):
    c = lax.axis_index("c")
    s = lax.axis_index("s")
    _zero_rows(zbuf, acc, s)

    def ostep(i, carry):
        r = i // 8
        j = i % 8
        onesbuf[r, pl.ds(j * 16, 16)] = jnp.ones((16,), F32)
        return carry

    lax.fori_loop(0, _CHD * 8, ostep, 0)
    plsc.subcore_barrier()
    base = (s * 2 + c) * _EWD

    def step(i, carry):
        off = base + i * _CHD
        pltpu.sync_copy(dstidx.at[pl.ds(off, _CHD)], idxv)
        pltpu.sync_copy(onesbuf, acc.at[idxv], add=True)
        return carry

    lax.fori_loop(0, _EWD // _CHD, step, 0)
    plsc.subcore_barrier()
    stripe = pl.ds(s * _NRS, _NRS)
    pltpu.sync_copy(acc.at[stripe], out.at[c, stripe])


def _sc_deg(dst):
    return pl.kernel(
        _sc_deg_body,
        out_type=jax.ShapeDtypeStruct((2, _NP, _H), F32),
        mesh=_mesh(),
        scratch_types=[
            pltpu.VMEM_SHARED((_NP, _H), F32),
            pltpu.VMEM((_CHD,), jnp.int32),
            pltpu.VMEM((_CHD, _H), F32),
            pltpu.VMEM((_ZR, _H), F32),
        ],
    )(dst)


def _sc_gather(ah, bh, vh, src, dst, half):
    return _make_gather(half)(ah, bh, vh, src, dst)


def _sc_scatter(pay, dst, half):
    return _make_scatter(half)(pay, dst)


def kernel(h, e, edge_index, Wn, bn, We, be, A, B, C, U, V,
           gh_g, gh_b, ge_g, ge_b, W1, b1, W2, b2, W3, b3):
    src = edge_index[0]
    dst = edge_index[1]

    hh = _mm_bias(h, Wn, bn, _NBLK)
    ee = _mm_bias(e, We, be, _EBLK)

    degacc = _sc_deg(dst)
    norm = _norm_from_deg(degacc)

    zstat = jnp.zeros((8, _H), F32)
    for i in range(_L):
        w4 = jnp.stack([A[i], B[i], U[i], V[i]])
        nm = _node_mm(hh, w4)
        g0, v0 = _sc_gather(nm[0], nm[1], nm[3], src, dst, 0)
        en0, s10, s20 = _edge_a(ee, 0, g0, C[i])
        gam = ge_g[i].reshape(1, _H)
        bet = ge_b[i].reshape(1, _H)
        ee, pay0 = _edge_b(en0, ee, 0, v0,
                           s10, zstat, s20, zstat, gam, bet)
        acc0 = _sc_scatter(pay0, dst, 0)
        hn, ns1, ns2 = _node_a(nm, acc0, norm)
        hh, hsum = _node_b(hn, hh, ns1, ns2,
                           gh_g[i].reshape(1, _H), gh_b[i].reshape(1, _H))

    h2 = _H // 2
    h4 = _H // 4
    nc = W3.shape[1]
    w1p = jnp.zeros((_H, _H), F32).at[:, :h2].set(W1)
    b1p = jnp.zeros((1, _H), F32).at[0, :h2].set(b1)
    w2p = jnp.zeros((_H, _H), F32).at[:h2, :h4].set(W2)
    b2p = jnp.zeros((1, _H), F32).at[0, :h4].set(b2)
    w3p = jnp.zeros((_H, _H), F32).at[:h4, :nc].set(W3)
    b3p = jnp.zeros((1, _H), F32).at[0, :nc].set(b3)
    out = _readout(hsum, w1p, b1p, w2p, b2p, w3p, b3p)
    return out[0:1, 0:nc]


# speedup vs baseline: 1.6515x; 1.0605x over previous
import jax
import jax.numpy as jnp
from jax import lax
from jax.experimental import pallas as pl
from jax.experimental.pallas import tpu as pltpu
from jax.experimental.pallas import tpu_sc as plsc

F32 = jnp.float32

_N = 10000
_E = 320000
_E2 = _E
_H = 128
_L = 4

_EBLK = 8000
_E2BLKS = _E2 // _EBLK
_NBLK = 2000
_NW = 32
_NP = 10240
_NRS = _NP // 16
_ZR = 16

_CHG = 80
_EWG = _E2 // _NW
_CHD = 40
_EWD = _E // _NW
_CHS = 80
_ESS = _E2 // 16


def _mm_bias(x, w, b, blk):
    m = x.shape[0]

    def body(x_ref, w_ref, b_ref, o_ref):
        o_ref[...] = (
            jnp.dot(x_ref[...], w_ref[...], preferred_element_type=F32)
            + b_ref[...]
        )

    return pl.pallas_call(
        body,
        grid=(m // blk,),
        in_specs=[
            pl.BlockSpec((blk, _H), lambda i: (i, 0)),
            pl.BlockSpec((_H, _H), lambda i: (0, 0)),
            pl.BlockSpec((1, _H), lambda i: (0, 0)),
        ],
        out_specs=pl.BlockSpec((blk, _H), lambda i: (i, 0)),
        out_shape=jax.ShapeDtypeStruct((m, _H), F32),
    )(x, w, b.reshape(1, _H))


def _node_mm(hh, w4):
    def body(h_ref, w_ref, o_ref):
        for k in range(4):
            o_ref[k] = jnp.dot(h_ref[...], w_ref[k], preferred_element_type=F32)

    return pl.pallas_call(
        body,
        grid=(_N // _NBLK,),
        in_specs=[
            pl.BlockSpec((_NBLK, _H), lambda i: (i, 0)),
            pl.BlockSpec((4, _H, _H), lambda i: (0, 0, 0)),
        ],
        out_specs=pl.BlockSpec((4, _NBLK, _H), lambda i: (0, i, 0)),
        out_shape=jax.ShapeDtypeStruct((4, _N, _H), F32),
    )(hh, w4)


def _edge_a(ee, off, g, c):
    def body(ee_ref, g_ref, c_ref, en_ref, s1_ref, s2_ref):
        en = (
            jnp.dot(ee_ref[...], c_ref[...], preferred_element_type=F32)
            + g_ref[...]
        )
        en_ref[...] = en

        @pl.when(pl.program_id(0) == 0)
        def _():
            s1_ref[...] = jnp.zeros_like(s1_ref)
            s2_ref[...] = jnp.zeros_like(s2_ref)

        s1_ref[...] += jnp.broadcast_to(
            jnp.sum(en, axis=0, keepdims=True), (8, _H))
        s2_ref[...] += jnp.broadcast_to(
            jnp.sum(en * en, axis=0, keepdims=True), (8, _H))

    return pl.pallas_call(
        body,
        grid=(_E2BLKS,),
        in_specs=[
            pl.BlockSpec((_EBLK, _H), lambda i: (i + off, 0)),
            pl.BlockSpec((_EBLK, _H), lambda i: (i, 0)),
            pl.BlockSpec((_H, _H), lambda i: (0, 0)),
        ],
        out_specs=[
            pl.BlockSpec((_EBLK, _H), lambda i: (i, 0)),
            pl.BlockSpec((8, _H), lambda i: (0, 0)),
            pl.BlockSpec((8, _H), lambda i: (0, 0)),
        ],
        out_shape=[
            jax.ShapeDtypeStruct((_E2, _H), F32),
            jax.ShapeDtypeStruct((8, _H), F32),
            jax.ShapeDtypeStruct((8, _H), F32),
        ],
    )(ee, g, c)


def _edge_b(en, ee, off, vs, s1a, s1b, s2a, s2b, gam, bet):
    def body(en_ref, ee_ref, vs_ref, s1a_ref, s1b_ref, s2a_ref, s2b_ref,
             g_ref, b_ref, eo_ref, p_ref):
        mu = (s1a_ref[0:1, :] + s1b_ref[0:1, :]) * (1.0 / _E)
        var = (s2a_ref[0:1, :] + s2b_ref[0:1, :]) * (1.0 / _E) - mu * mu
        x = en_ref[...]
        xn = g_ref[...] * (x - mu) * lax.rsqrt(var + 1e-5) + b_ref[...]
        ea = jnp.maximum(xn, 0.0)
        sig = jax.nn.sigmoid(ea)
        eo_ref[...] = ee_ref[...] + ea
        sv = sig * vs_ref[...]
        p_ref[0] = jnp.concatenate([sig[:, :64], sv[:, :64]], axis=1)
        p_ref[1] = jnp.concatenate([sig[:, 64:], sv[:, 64:]], axis=1)

    stat = pl.BlockSpec((8, _H), lambda i: (0, 0))
    vec = pl.BlockSpec((1, _H), lambda i: (0, 0))
    return pl.pallas_call(
        body,
        grid=(_E2BLKS,),
        in_specs=[
            pl.BlockSpec((_EBLK, _H), lambda i: (i, 0)),
            pl.BlockSpec((_EBLK, _H), lambda i: (i + off, 0)),
            pl.BlockSpec((_EBLK, _H), lambda i: (i, 0)),
            stat, stat, stat, stat, vec, vec,
        ],
        out_specs=[
            pl.BlockSpec((_EBLK, _H), lambda i: (i, 0)),
            pl.BlockSpec((2, _EBLK, _H), lambda i: (0, i, 0)),
        ],
        out_shape=[
            jax.ShapeDtypeStruct((_E2, _H), F32),
            jax.ShapeDtypeStruct((2, _E2, _H), F32),
        ],
    )(en, ee, vs, s1a, s1b, s2a, s2b, gam, bet)


def _norm_from_deg(degacc):
    def body(d_ref, n_ref):
        deg = d_ref[0, :, 0:1] + d_ref[1, :, 0:1]
        nv = lax.rsqrt(jnp.maximum(deg, 1.0))
        n_ref[...] = jnp.broadcast_to(nv, (_NP, _H))

    return pl.pallas_call(
        body, out_shape=jax.ShapeDtypeStruct((_NP, _H), F32))(degacc)


def _node_a(nm, acc0, norm):
    def body(u_ref, a0_ref, no_ref, hn_ref, s1_ref, s2_ref):
        a = a0_ref[...]
        ss = jnp.concatenate([a[0, :, :64], a[1, :, :64]], axis=1)
        sh = jnp.concatenate([a[0, :, 64:], a[1, :, 64:]], axis=1)
        hnew = u_ref[0] + no_ref[...] * (sh / (ss + 1e-6))
        hn_ref[...] = hnew

        @pl.when(pl.program_id(0) == 0)
        def _():
            s1_ref[...] = jnp.zeros_like(s1_ref)
            s2_ref[...] = jnp.zeros_like(s2_ref)

        s1_ref[...] += jnp.broadcast_to(
            jnp.sum(hnew, axis=0, keepdims=True), (8, _H))
        s2_ref[...] += jnp.broadcast_to(
            jnp.sum(hnew * hnew, axis=0, keepdims=True), (8, _H))

    acc_spec = pl.BlockSpec((2, _NBLK, _H), lambda i: (0, i, 0))
    return pl.pallas_call(
        body,
        grid=(_N // _NBLK,),
        in_specs=[
            pl.BlockSpec((1, _NBLK, _H), lambda i: (2, i, 0)),
            acc_spec,
            pl.BlockSpec((_NBLK, _H), lambda i: (i, 0)),
        ],
        out_specs=[
            pl.BlockSpec((_NBLK, _H), lambda i: (i, 0)),
            pl.BlockSpec((8, _H), lambda i: (0, 0)),
            pl.BlockSpec((8, _H), lambda i: (0, 0)),
        ],
        out_shape=[
            jax.ShapeDtypeStruct((_N, _H), F32),
            jax.ShapeDtypeStruct((8, _H), F32),
            jax.ShapeDtypeStruct((8, _H), F32),
        ],
    )(nm, acc0, norm)


def _node_b(hn, hin, s1, s2, gam, bet):
    def body(hn_ref, hi_ref, s1_ref, s2_ref, g_ref, b_ref, ho_ref, hs_ref):
        mu = s1_ref[0:1, :] * (1.0 / _N)
        var = s2_ref[0:1, :] * (1.0 / _N) - mu * mu
        x = hn_ref[...]
        ha = jnp.maximum(
            g_ref[...] * (x - mu) * lax.rsqrt(var + 1e-5) + b_ref[...], 0.0)
        hh = hi_ref[...] + ha
        ho_ref[...] = hh

        @pl.when(pl.program_id(0) == 0)
        def _():
            hs_ref[...] = jnp.zeros_like(hs_ref)

        hs_ref[...] += jnp.broadcast_to(
            jnp.sum(hh, axis=0, keepdims=True), (8, _H))

    return pl.pallas_call(
        body,
        grid=(_N // _NBLK,),
        in_specs=[
            pl.BlockSpec((_NBLK, _H), lambda i: (i, 0)),
            pl.BlockSpec((_NBLK, _H), lambda i: (i, 0)),
            pl.BlockSpec((8, _H), lambda i: (0, 0)),
            pl.BlockSpec((8, _H), lambda i: (0, 0)),
            pl.BlockSpec((1, _H), lambda i: (0, 0)),
            pl.BlockSpec((1, _H), lambda i: (0, 0)),
        ],
        out_specs=[
            pl.BlockSpec((_NBLK, _H), lambda i: (i, 0)),
            pl.BlockSpec((8, _H), lambda i: (0, 0)),
        ],
        out_shape=[
            jax.ShapeDtypeStruct((_N, _H), F32),
            jax.ShapeDtypeStruct((8, _H), F32),
        ],
    )(hn, hin, s1, s2, gam, bet)


def _readout(hsum, w1, b1, w2, b2, w3, b3):
    def body(hs_ref, w1_ref, b1_ref, w2_ref, b2_ref, w3_ref, b3_ref, o_ref):
        hg = hs_ref[...] * (1.0 / _N)
        r = jnp.maximum(
            jnp.dot(hg, w1_ref[...], preferred_element_type=F32)
            + b1_ref[...], 0.0)
        r = jnp.maximum(
            jnp.dot(r, w2_ref[...], preferred_element_type=F32)
            + b2_ref[...], 0.0)
        o_ref[...] = (
            jnp.dot(r, w3_ref[...], preferred_element_type=F32) + b3_ref[...])

    return pl.pallas_call(
        body, out_shape=jax.ShapeDtypeStruct((8, _H), F32))(
            hsum, w1, b1, w2, b2, w3, b3)


def _mesh():
    return plsc.VectorSubcoreMesh(
        core_axis_name="c", subcore_axis_name="s",
        num_cores=2, num_subcores=16)


def _zero_rows(zbuf, acc, s):
    def zstep(i, carry):
        r = i // 8
        j = i % 8
        zbuf[r, pl.ds(j * 16, 16)] = jnp.zeros((16,), F32)
        return carry

    lax.fori_loop(0, _ZR * 8, zstep, 0)
    for k in range(_NRS // _ZR):
        pltpu.sync_copy(zbuf, acc.at[pl.ds(s * _NRS + k * _ZR, _ZR)])


def _make_gather(half):
    def body(ah, bh, vh, src, dst, g_out, v_out,
             idx1, idx2, bufg, bufv, sem, sem2, sem3):
        wid = lax.axis_index("s") * 2 + lax.axis_index("c")
        base_l = wid * _EWG

        def step(i, carry):
            off_l = base_l + i * _CHG
            off_g = half * _E2 + off_l
            c1 = pltpu.async_copy(src.at[pl.ds(off_g, _CHG)], idx1, sem)
            c2 = pltpu.async_copy(dst.at[pl.ds(off_g, _CHG)], idx2, sem2)
            for r in range(_CHG):
                for j in range(8):
                    bufg[r, pl.ds(j * 16, 16)] = jnp.zeros((16,), F32)
            c1.wait()
            c2.wait()
            ca = pltpu.async_copy(ah.at[idx1], bufg, sem, add=True)
            cb = pltpu.async_copy(bh.at[idx2], bufg, sem3, add=True)
            cv = pltpu.async_copy(vh.at[idx1], bufv, sem2)
            ca.wait()
            cb.wait()
            cv.wait()
            cg = pltpu.async_copy(bufg, g_out.at[pl.ds(off_l, _CHG)], sem)
            cw = pltpu.async_copy(bufv, v_out.at[pl.ds(off_l, _CHG)], sem2)
            cg.wait()
            cw.wait()
            return carry

        lax.fori_loop(0, _EWG // _CHG, step, 0)

    return pl.kernel(
        body,
        out_type=(
            jax.ShapeDtypeStruct((_E2, _H), F32),
            jax.ShapeDtypeStruct((_E2, _H), F32),
        ),
        mesh=_mesh(),
        scratch_types=[
            pltpu.VMEM((_CHG,), jnp.int32),
            pltpu.VMEM((_CHG,), jnp.int32),
            pltpu.VMEM((_CHG, _H), F32),
            pltpu.VMEM((_CHG, _H), F32),
            pltpu.SemaphoreType.DMA,
            pltpu.SemaphoreType.DMA,
            pltpu.SemaphoreType.DMA,
        ],
    )


def _make_scatter(half):
    def body(pay, dstidx, out, acc, idxv, buf, zbuf, sem, sem2):
        c = lax.axis_index("c")
        s = lax.axis_index("s")
        _zero_rows(zbuf, acc, s)
        plsc.subcore_barrier()
        base_l = s * _ESS

        def step(i, carry):
            off_l = base_l + i * _CHS
            off_g = half * _E2 + off_l
            ci = pltpu.async_copy(dstidx.at[pl.ds(off_g, _CHS)], idxv, sem)
            cp = pltpu.async_copy(pay.at[c, pl.ds(off_l, _CHS)], buf, sem2)
            ci.wait()
            cp.wait()
            pltpu.sync_copy(buf, acc.at[idxv], add=True)
            return carry

        lax.fori_loop(0, _ESS // _CHS, step, 0)
        plsc.subcore_barrier()
        stripe = pl.ds(s * _NRS, _NRS)
        pltpu.sync_copy(acc.at[stripe], out.at[c, stripe])

    return pl.kernel(
        body,
        out_type=jax.ShapeDtypeStruct((2, _NP, _H), F32),
        mesh=_mesh(),
        scratch_types=[
            pltpu.VMEM_SHARED((_NP, _H), F32),
            pltpu.VMEM((_CHS,), jnp.int32),
            pltpu.VMEM((_CHS, _H), F32),
            pltpu.VMEM((_ZR, _H), F32),
            pltpu.SemaphoreType.DMA,
            pltpu.SemaphoreType.DMA,
        ],
    )


def _sc_deg_body(dstidx, out, acc, idxv, onesbuf, zbuf):
    c = lax.axis_index("c")
    s = lax.axis_index("s")
    _zero_rows(zbuf, acc, s)

    def ostep(i, carry):
        r = i // 8
        j = i % 8
        onesbuf[r, pl.ds(j * 16, 16)] = jnp.ones((16,), F32)
        return carry

    lax.fori_loop(0, _CHD * 8, ostep, 0)
    plsc.subcore_barrier()
    base = (s * 2 + c) * _EWD

    def step(i, carry):
        off = base + i * _CHD
        pltpu.sync_copy(dstidx.at[pl.ds(off, _CHD)], idxv)
        pltpu.sync_copy(onesbuf, acc.at[idxv], add=True)
        return carry

    lax.fori_loop(0, _EWD // _CHD, step, 0)
    plsc.subcore_barrier()
    stripe = pl.ds(s * _NRS, _NRS)
    pltpu.sync_copy(acc.at[stripe], out.at[c, stripe])


def _sc_deg(dst):
    return pl.kernel(
        _sc_deg_body,
        out_type=jax.ShapeDtypeStruct((2, _NP, _H), F32),
        mesh=_mesh(),
        scratch_types=[
            pltpu.VMEM_SHARED((_NP, _H), F32),
            pltpu.VMEM((_CHD,), jnp.int32),
            pltpu.VMEM((_CHD, _H), F32),
            pltpu.VMEM((_ZR, _H), F32),
        ],
    )(dst)


def _sc_gather(ah, bh, vh, src, dst, half):
    return _make_gather(half)(ah, bh, vh, src, dst)


def _sc_scatter(pay, dst, half):
    return _make_scatter(half)(pay, dst)


def kernel(h, e, edge_index, Wn, bn, We, be, A, B, C, U, V,
           gh_g, gh_b, ge_g, ge_b, W1, b1, W2, b2, W3, b3):
    src = edge_index[0]
    dst = edge_index[1]

    hh = _mm_bias(h, Wn, bn, _NBLK)
    ee = _mm_bias(e, We, be, _EBLK)

    degacc = _sc_deg(dst)
    norm = _norm_from_deg(degacc)

    zstat = jnp.zeros((8, _H), F32)
    for i in range(_L):
        w4 = jnp.stack([A[i], B[i], U[i], V[i]])
        nm = _node_mm(hh, w4)
        g0, v0 = _sc_gather(nm[0], nm[1], nm[3], src, dst, 0)
        en0, s10, s20 = _edge_a(ee, 0, g0, C[i])
        gam = ge_g[i].reshape(1, _H)
        bet = ge_b[i].reshape(1, _H)
        ee, pay0 = _edge_b(en0, ee, 0, v0,
                           s10, zstat, s20, zstat, gam, bet)
        acc0 = _sc_scatter(pay0, dst, 0)
        hn, ns1, ns2 = _node_a(nm, acc0, norm)
        hh, hsum = _node_b(hn, hh, ns1, ns2,
                           gh_g[i].reshape(1, _H), gh_b[i].reshape(1, _H))

    h2 = _H // 2
    h4 = _H // 4
    nc = W3.shape[1]
    w1p = jnp.zeros((_H, _H), F32).at[:, :h2].set(W1)
    b1p = jnp.zeros((1, _H), F32).at[0, :h2].set(b1)
    w2p = jnp.zeros((_H, _H), F32).at[:h2, :h4].set(W2)
    b2p = jnp.zeros((1, _H), F32).at[0, :h4].set(b2)
    w3p = jnp.zeros((_H, _H), F32).at[:h4, :nc].set(W3)
    b3p = jnp.zeros((1, _H), F32).at[0, :nc].set(b3)
    out = _readout(hsum, w1p, b1p, w2p, b2p, w3p, b3p)
    return out[0:1, 0:nc]
